# Initial kernel scaffold; baseline (speedup 1.0000x reference)
import jax
import jax.numpy as jnp
from jax.experimental import pallas as pl


def kernel(x, edge_index, Wq0, bq0, Wk0, bk0, W0, b0, Wq1, bq1, Wk1, bk1, W1, b1):
    raise NotImplementedError("write your pallas kernel here")


# speedup vs baseline: 18.2157x; 18.2157x over previous
import jax
import jax.numpy as jnp
from jax import lax
from jax.experimental import pallas as pl
from jax.experimental.pallas import tpu as pltpu
from jax.experimental.pallas import tpu_sc as plsc

N = 10000
DF = 128
E_RAW = 320000
E_TOT = E_RAW + N
W = 128
E_PAD = 331776
NW = 32
RPT = N // 16

_HIGH = lax.Precision.HIGHEST


def _mesh():
    return plsc.VectorSubcoreMesh(core_axis_name="c", subcore_axis_name="s")


_SC_PARAMS = pltpu.CompilerParams(use_tc_tiling_on_sc=False)


def _sc_gather(tables, idxs):
    n = len(tables)
    out_type = tuple(
        jax.ShapeDtypeStruct((E_PAD, t.shape[1]), t.dtype) for t in tables
    )

    @pl.kernel(out_type=out_type, mesh=_mesh(), compiler_params=_SC_PARAMS)
    def gat_kernel(*refs):
        tbl = refs[:n]
        idx = refs[n:2 * n]
        outs = refs[2 * n:]

        def body(*bufs):
            ivs = bufs[:n]
            ovs = bufs[n:]
            for t in range(n):
                pltpu.sync_copy(tbl[t].at[ivs[t].at[0]], ovs[t])

        pltpu.emit_pipeline(
            body,
            grid=(E_PAD // W,),
            in_specs=[pl.BlockSpec((1, W), lambda i: (0, i)) for _ in range(n)],
            out_specs=[
                pl.BlockSpec((W, tables[t].shape[1]), lambda i: (i, 0))
                for t in range(n)
            ],
            core_axis_name=("c", "s"),
            dimension_semantics=(pltpu.PARALLEL,),
        )(*idx, *outs)

    return gat_kernel(*tables, *idxs)


def _sc_scatter_add(src, idx, zrows, chunk):
    d = src.shape[1]
    per_w = E_PAD // NW
    n_chunks = per_w // chunk

    @pl.kernel(
        out_type=jax.ShapeDtypeStruct((2, N, d), jnp.float32),
        mesh=_mesh(),
        compiler_params=_SC_PARAMS,
        scratch_types=[
            pltpu.VMEM((chunk,), jnp.int32),
            pltpu.VMEM((chunk, d), jnp.float32),
            pltpu.VMEM_SHARED((N, d), jnp.float32),
        ],
    )
    def sct_kernel(src_hbm, idx_hbm, z_hbm, out_hbm, idx_v, src_v, acc_sh):
        c = lax.axis_index("c")
        s = lax.axis_index("s")
        wid = s * 2 + c
        rows = pl.ds(s * RPT, RPT)
        pltpu.sync_copy(z_hbm, acc_sh.at[rows])
        plsc.subcore_barrier()

        @pl.loop(0, n_chunks)
        def _(g):
            base = wid * per_w + g * chunk
            pltpu.sync_copy(idx_hbm.at[pl.ds(base, chunk)], idx_v)
            pltpu.sync_copy(src_hbm.at[pl.ds(base, chunk)], src_v)
            pltpu.sync_copy(src_v, acc_sh.at[idx_v], add=True)

        plsc.subcore_barrier()
        pltpu.sync_copy(acc_sh.at[rows], out_hbm.at[c, rows])

    return sct_kernel(src, idx, zrows)


def _tc_qkv(x, Wq, bq, Wk, bk, Wv):
    blk = 2000

    def body(x_ref, wq_ref, bq_ref, wk_ref, bk_ref, wv_ref, q_ref, k_ref, v_ref):
        xb = x_ref[...]
        q_ref[...] = jax.nn.relu(
            jnp.dot(xb, wq_ref[...], preferred_element_type=jnp.float32,
                    precision=_HIGH) + bq_ref[...])
        k_ref[...] = jax.nn.relu(
            jnp.dot(xb, wk_ref[...], preferred_element_type=jnp.float32,
                    precision=_HIGH) + bk_ref[...])
        v_ref[...] = jnp.dot(xb, wv_ref[...], preferred_element_type=jnp.float32,
                             precision=_HIGH)

    full = lambda i: (0, 0)
    o64 = jax.ShapeDtypeStruct((N, 64), jnp.float32)
    return pl.pallas_call(
        body,
        grid=(N // blk,),
        in_specs=[
            pl.BlockSpec((blk, DF), lambda i: (i, 0)),
            pl.BlockSpec((DF, 64), full),
            pl.BlockSpec((1, 64), full),
            pl.BlockSpec((DF, 64), full),
            pl.BlockSpec((1, 64), full),
            pl.BlockSpec((DF, 64), full),
        ],
        out_specs=[pl.BlockSpec((blk, 64), lambda i: (i, 0))] * 3,
        out_shape=[o64, o64, o64],
    )(x, Wq, bq.reshape(1, 64), Wk, bk.reshape(1, 64), Wv)


def _tc_p0(qe, ke, heads):
    blk = 4096

    def body(qe_ref, ke_ref, s_ref, p_ref):
        i = pl.program_id(0)
        prod = qe_ref[...] * ke_ref[...]
        att = jnp.dot(prod, s_ref[...], preferred_element_type=jnp.float32,
                      precision=_HIGH)
        eid = i * blk + lax.broadcasted_iota(jnp.int32, (blk, 8), 0)
        p_ref[...] = jnp.where(eid < E_TOT, jnp.exp(att), 0.0)

    return pl.pallas_call(
        body,
        grid=(E_PAD // blk,),
        in_specs=[
            pl.BlockSpec((blk, 64), lambda i: (i, 0)),
            pl.BlockSpec((blk, 64), lambda i: (i, 0)),
            pl.BlockSpec((64, 8), lambda i: (0, 0)),
        ],
        out_specs=pl.BlockSpec((blk, 8), lambda i: (i, 0)),
        out_shape=jax.ShapeDtypeStruct((E_PAD, 8), jnp.float32),
    )(qe, ke, heads)


def _tc_contrib0(p, dga, dgb, ve, rep):
    blk = 4096

    def body(p_ref, da_ref, db_ref, ve_ref, rep_ref, o_ref):
        alpha = p_ref[...] / (da_ref[...] + db_ref[...] + 1e-16)
        alpha64 = jnp.dot(alpha, rep_ref[...], preferred_element_type=jnp.float32,
                          precision=_HIGH)
        o_ref[...] = ve_ref[...] * alpha64

    return pl.pallas_call(
        body,
        grid=(E_PAD // blk,),
        in_specs=[
            pl.BlockSpec((blk, 8), lambda i: (i, 0)),
            pl.BlockSpec((blk, 8), lambda i: (i, 0)),
            pl.BlockSpec((blk, 8), lambda i: (i, 0)),
            pl.BlockSpec((blk, 64), lambda i: (i, 0)),
            pl.BlockSpec((8, 64), lambda i: (0, 0)),
        ],
        out_specs=pl.BlockSpec((blk, 64), lambda i: (i, 0)),
        out_shape=jax.ShapeDtypeStruct((E_PAD, 64), jnp.float32),
    )(p, dga, dgb, ve, rep)


def _tc_layer1_tables(oa, ob, b0, Wq1b, bq1b, Wk1b, bk1b, W1p):
    blk = 2000

    def body(a_ref, b_ref, b0_ref, wq_ref, bq_ref, wk_ref, bk_ref, wv_ref,
             q_ref, k_ref, v_ref):
        h = jax.nn.relu(a_ref[...] + b_ref[...] + b0_ref[...])
        q_ref[...] = jax.nn.relu(
            jnp.dot(h, wq_ref[...], preferred_element_type=jnp.float32,
                    precision=_HIGH) + bq_ref[...])
        k_ref[...] = jax.nn.relu(
            jnp.dot(h, wk_ref[...], preferred_element_type=jnp.float32,
                    precision=_HIGH) + bk_ref[...])
        v_ref[...] = jnp.dot(h, wv_ref[...], preferred_element_type=jnp.float32,
                             precision=_HIGH)

    full = lambda i: (0, 0)
    o8 = jax.ShapeDtypeStruct((N, 8), jnp.float32)
    return pl.pallas_call(
        body,
        grid=(N // blk,),
        in_specs=[
            pl.BlockSpec((blk, 64), lambda i: (i, 0)),
            pl.BlockSpec((blk, 64), lambda i: (i, 0)),
            pl.BlockSpec((1, 64), full),
            pl.BlockSpec((64, 8), full),
            pl.BlockSpec((1, 8), full),
            pl.BlockSpec((64, 8), full),
            pl.BlockSpec((1, 8), full),
            pl.BlockSpec((64, 8), full),
        ],
        out_specs=[pl.BlockSpec((blk, 8), lambda i: (i, 0))] * 3,
        out_shape=[o8, o8, o8],
    )(oa, ob, b0.reshape(1, 64), Wq1b, bq1b, Wk1b, bk1b, W1p)


def _tc_p1(q1e, k1e):
    blk = 4096

    def body(q_ref, k_ref, p_ref):
        i = pl.program_id(0)
        eid = i * blk + lax.broadcasted_iota(jnp.int32, (blk, 8), 0)
        p_ref[...] = jnp.where(eid < E_TOT, jnp.exp(q_ref[...] * k_ref[...]), 0.0)

    return pl.pallas_call(
        body,
        grid=(E_PAD // blk,),
        in_specs=[pl.BlockSpec((blk, 8), lambda i: (i, 0))] * 2,
        out_specs=pl.BlockSpec((blk, 8), lambda i: (i, 0)),
        out_shape=jax.ShapeDtypeStruct((E_PAD, 8), jnp.float32),
    )(q1e, k1e)


def _tc_contrib1(p1, d1a, d1b, v1e):
    blk = 4096

    def body(p_ref, da_ref, db_ref, v_ref, o_ref):
        alpha = p_ref[...] / (da_ref[...] + db_ref[...] + 1e-16)
        o_ref[...] = v_ref[...] * alpha

    return pl.pallas_call(
        body,
        grid=(E_PAD // blk,),
        in_specs=[pl.BlockSpec((blk, 8), lambda i: (i, 0))] * 4,
        out_specs=pl.BlockSpec((blk, 8), lambda i: (i, 0)),
        out_shape=jax.ShapeDtypeStruct((E_PAD, 8), jnp.float32),
    )(p1, d1a, d1b, v1e)


def _tc_final(o1a, o1b, b1p):
    blk = 2000

    def body(a_ref, b_ref, bias_ref, o_ref):
        o_ref[...] = a_ref[...] + b_ref[...] + bias_ref[...]

    return pl.pallas_call(
        body,
        grid=(N // blk,),
        in_specs=[
            pl.BlockSpec((blk, 8), lambda i: (i, 0)),
            pl.BlockSpec((blk, 8), lambda i: (i, 0)),
            pl.BlockSpec((1, 8), lambda i: (0, 0)),
        ],
        out_specs=pl.BlockSpec((blk, 8), lambda i: (i, 0)),
        out_shape=jax.ShapeDtypeStruct((N, 8), jnp.float32),
    )(o1a, o1b, b1p)


def kernel(x, edge_index, Wq0, bq0, Wk0, bk0, W0, b0, Wq1, bq1, Wk1, bk1, W1, b1):
    loops = jnp.arange(N, dtype=jnp.int32)
    pad = jnp.zeros((E_PAD - E_TOT,), jnp.int32)
    row = jnp.concatenate([edge_index[0].astype(jnp.int32), loops, pad])
    col = jnp.concatenate([edge_index[1].astype(jnp.int32), loops, pad])
    row2 = row.reshape(1, E_PAD)
    col2 = col.reshape(1, E_PAD)

    heads = jnp.repeat(jnp.eye(8, dtype=jnp.float32), 8, axis=0)
    rep = heads.T
    z8 = jnp.zeros((RPT, 8), jnp.float32)
    z64 = jnp.zeros((RPT, 64), jnp.float32)

    Q, K, V = _tc_qkv(x, Wq0, bq0, Wk0, bk0, W0)
    qe, ke = _sc_gather([Q, K], [row2, col2])
    p = _tc_p0(qe, ke, heads)
    dpart = _sc_scatter_add(p, row, z8, 1296)
    dga, dgb, ve = _sc_gather([dpart[0], dpart[1], V], [row2, row2, col2])
    contrib = _tc_contrib0(p, dga, dgb, ve, rep)
    opart = _sc_scatter_add(contrib, row, z64, 648)

    Wq1b = jnp.broadcast_to(Wq1, (64, 8))
    bq1b = jnp.broadcast_to(bq1, (1, 8))
    Wk1b = jnp.broadcast_to(Wk1, (64, 8))
    bk1b = jnp.broadcast_to(bk1, (1, 8))
    W1p = jnp.pad(W1, ((0, 0), (0, 1)))
    q1t, k1t, v1t = _tc_layer1_tables(opart[0], opart[1], b0,
                                      Wq1b, bq1b, Wk1b, bk1b, W1p)
    q1e, k1e = _sc_gather([q1t, k1t], [row2, col2])
    p1 = _tc_p1(q1e, k1e)
    d1part = _sc_scatter_add(p1, row, z8, 1296)
    d1a, d1b, v1e = _sc_gather([d1part[0], d1part[1], v1t], [row2, row2, col2])
    contrib1 = _tc_contrib1(p1, d1a, d1b, v1e)
    o1part = _sc_scatter_add(contrib1, row, z8, 1296)
    out8 = _tc_final(o1part[0], o1part[1], jnp.pad(b1, (0, 1)).reshape(1, 8))
    return out8[:, :7]


# speedup vs baseline: 56.3161x; 3.0916x over previous
import jax
import jax.numpy as jnp
from jax import lax
from jax.experimental import pallas as pl
from jax.experimental.pallas import tpu as pltpu
from jax.experimental.pallas import tpu_sc as plsc

N = 10000
DF = 128
E_RAW = 320000
E_TOT = E_RAW + N
E_PAD = 331776
NW = 32
N_ACC = 10016
RPT = N_ACC // 16

_HIGH = lax.Precision.HIGHEST


def _mesh():
    return plsc.VectorSubcoreMesh(core_axis_name="c", subcore_axis_name="s")


_SC_PARAMS = pltpu.CompilerParams(use_tc_tiling_on_sc=False,
                                  needs_layout_passes=False)


def _vperm(v, idx):
    return lax.gather(
        v, idx.reshape(16, 1),
        lax.GatherDimensionNumbers(offset_dims=(), collapsed_slice_dims=(0,),
                                   start_index_map=(0,)),
        (1,), mode=lax.GatherScatterMode.PROMISE_IN_BOUNDS)


def _sc_attn0(q, k, row, col):
    ca = 288
    per_w = E_PAD // NW
    n_ch = per_w // ca

    i32 = jnp.int32
    f32 = jnp.float32
    scr = ([pltpu.VMEM((ca,), i32)] * 2 +
           [pltpu.VMEM((ca, 64), f32)] * 2 +
           [pltpu.VMEM((ca, 8), f32)])

    @pl.kernel(out_type=jax.ShapeDtypeStruct((E_PAD, 8), f32),
               mesh=_mesh(), compiler_params=_SC_PARAMS, scratch_types=scr)
    def attn_kernel(q_hbm, k_hbm, row_hbm, col_hbm, p_hbm,
                    ir_v, ic_v, qe_v, ke_v, pv_v):
        c = lax.axis_index("c")
        s = lax.axis_index("s")
        base_w = (s * 2 + c) * per_w
        io16 = lax.iota(i32, 16)
        p8 = io16 ^ 8
        h8 = io16 & 7
        lo8 = io16 < 8

        @pl.loop(0, n_ch)
        def _(ch):
            base = base_w + ch * ca
            pltpu.sync_copy(row_hbm.at[pl.ds(base, ca)], ir_v)
            pltpu.sync_copy(col_hbm.at[pl.ds(base, ca)], ic_v)
            pltpu.sync_copy(q_hbm.at[ir_v], qe_v)
            pltpu.sync_copy(k_hbm.at[ic_v], ke_v)

            @plsc.parallel_loop(0, ca, unroll=4)
            def _(e):
                acc = None
                for j in range(4):
                    qv = qe_v[e, pl.ds(j * 16, 16)]
                    kv = ke_v[e, pl.ds(j * 16, 16)]
                    pr = qv * kv
                    acc = pr if acc is None else acc + pr
                att = acc + _vperm(acc, p8)
                t = jnp.exp(att)
                plsc.store_scatter(pv_v, [jnp.full((16,), e, i32), h8],
                                   t, mask=lo8)

            pltpu.sync_copy(pv_v, p_hbm.at[pl.ds(base, ca)])

    return attn_kernel(q, k, row, col)


def _sc_aggr0(p, v, row_s, col, z8, z64):
    cb = 288
    per_w = E_PAD // NW
    n4 = per_w // cb
    c3 = 648
    p3w = E_PAD // 16
    n3 = p3w // c3

    i32 = jnp.int32
    f32 = jnp.float32
    scr = ([pltpu.VMEM((c3,), i32)] + [pltpu.VMEM((c3, 8), f32)] +
           [pltpu.VMEM((cb,), i32)] * 2 +
           [pltpu.VMEM((cb, 8), f32)] * 2 + [pltpu.VMEM((cb, 64), f32)] * 2 +
           [pltpu.VMEM_SHARED((N_ACC, 8), f32),
            pltpu.VMEM_SHARED((N_ACC, 64), f32)])

    @pl.kernel(out_type=(jax.ShapeDtypeStruct((2, N_ACC, 64), f32),
                         jax.ShapeDtypeStruct((2, N_ACC, 8), f32)),
               mesh=_mesh(), compiler_params=_SC_PARAMS, scratch_types=scr)
    def aggr_kernel(p_hbm, v_hbm, rs_hbm, col_hbm, z8_hbm, z64_hbm,
                    out_hbm, dh_hbm,
                    i3_v, p3_v, ir_v, ic_v, pv_v, dg_v, vv_v, ct_v,
                    den, acc):
        c = lax.axis_index("c")
        s = lax.axis_index("s")
        base_w = (s * 2 + c) * per_w
        io16 = lax.iota(i32, 16)
        hi1 = jnp.where(io16 >= 8, 1, 0).astype(i32)

        rows = pl.ds(s * RPT, RPT)
        pltpu.sync_copy(z8_hbm, den.at[rows])
        pltpu.sync_copy(z64_hbm, acc.at[rows])
        plsc.subcore_barrier()

        base3 = s * p3w

        @pl.loop(0, n3)
        def _(ch):
            base = base3 + ch * c3
            pltpu.sync_copy(rs_hbm.at[pl.ds(base, c3)], i3_v)
            pltpu.sync_copy(p_hbm.at[pl.ds(base, c3)], p3_v)
            pltpu.sync_copy(p3_v, den.at[i3_v], add=True)

        plsc.subcore_barrier()
        pltpu.sync_copy(den.at[rows], dh_hbm.at[c, rows])
        plsc.subcore_barrier()

        @pl.loop(0, n4)
        def _(ch):
            base = base_w + ch * cb
            pltpu.sync_copy(rs_hbm.at[pl.ds(base, cb)], ir_v)
            pltpu.sync_copy(col_hbm.at[pl.ds(base, cb)], ic_v)
            pltpu.sync_copy(p_hbm.at[pl.ds(base, cb)], pv_v)
            pltpu.sync_copy(v_hbm.at[ic_v], vv_v)
            pltpu.sync_copy(dh_hbm.at[c].at[ir_v], dg_v)

            @plsc.parallel_loop(0, cb, unroll=4)
            def _(e):
                se = jnp.full((16,), e, i32)
                pv8 = plsc.load_gather(pv_v, [se, io16 & 7])
                dg8 = plsc.load_gather(dg_v, [se, io16 & 7])
                av = pv8 / (dg8 + 1e-16)
                for j in range(4):
                    avj = _vperm(av, jnp.full((16,), 2 * j, i32) + hi1)
                    sl = pl.ds(j * 16, 16)
                    ct_v[e, sl] = vv_v[e, sl] * avj

            pltpu.sync_copy(ct_v, acc.at[ir_v], add=True)

        plsc.subcore_barrier()
        pltpu.sync_copy(acc.at[rows], out_hbm.at[c, rows])

    return aggr_kernel(p, v, row_s, col, z8, z64)


def _sc_attn1(q1t, k1t, row, col):
    ca = 648
    per_w = E_PAD // NW
    n_ch = per_w // ca

    i32 = jnp.int32
    f32 = jnp.float32
    scr = ([pltpu.VMEM((ca,), i32)] * 2 + [pltpu.VMEM((ca, 8), f32)] * 3)

    @pl.kernel(out_type=jax.ShapeDtypeStruct((E_PAD, 8), f32),
               mesh=_mesh(), compiler_params=_SC_PARAMS, scratch_types=scr)
    def attn1_kernel(q_hbm, k_hbm, row_hbm, col_hbm, p_hbm,
                     ir_v, ic_v, qe_v, ke_v, pv_v):
        c = lax.axis_index("c")
        s = lax.axis_index("s")
        base_w = (s * 2 + c) * per_w
        io16 = lax.iota(i32, 16)
        h8 = io16 & 7
        hi1 = jnp.where(io16 >= 8, 1, 0).astype(i32)

        @pl.loop(0, n_ch)
        def _(ch):
            base = base_w + ch * ca
            pltpu.sync_copy(row_hbm.at[pl.ds(base, ca)], ir_v)
            pltpu.sync_copy(col_hbm.at[pl.ds(base, ca)], ic_v)
            pltpu.sync_copy(q_hbm.at[ir_v], qe_v)
            pltpu.sync_copy(k_hbm.at[ic_v], ke_v)

            @plsc.parallel_loop(0, ca // 2, unroll=8)
            def _(ee):
                e2 = jnp.full((16,), 2 * ee, i32) + hi1
                qv = plsc.load_gather(qe_v, [e2, h8])
                kv = plsc.load_gather(ke_v, [e2, h8])
                plsc.store_scatter(pv_v, [e2, h8], jnp.exp(qv * kv))

            pltpu.sync_copy(pv_v, p_hbm.at[pl.ds(base, ca)])

    return attn1_kernel(q1t, k1t, row, col)


def _sc_aggr1(p1, v1t, row_s, col, z8):
    cb = 648
    per_w = E_PAD // NW
    n4 = per_w // cb
    c3 = 648
    p3w = E_PAD // 16
    n3 = p3w // c3

    i32 = jnp.int32
    f32 = jnp.float32
    scr = ([pltpu.VMEM((c3,), i32)] + [pltpu.VMEM((c3, 8), f32)] +
           [pltpu.VMEM((cb,), i32)] * 2 +
           [pltpu.VMEM((cb, 8), f32)] * 4 +
           [pltpu.VMEM_SHARED((N_ACC, 8), f32),
            pltpu.VMEM_SHARED((N_ACC, 8), f32)])

    @pl.kernel(out_type=(jax.ShapeDtypeStruct((2, N_ACC, 8), f32),
                         jax.ShapeDtypeStruct((2, N_ACC, 8), f32)),
               mesh=_mesh(), compiler_params=_SC_PARAMS, scratch_types=scr)
    def aggr1_kernel(p_hbm, v_hbm, rs_hbm, col_hbm, z8_hbm,
                     out_hbm, dh_hbm,
                     i3_v, p3_v, ir_v, ic_v, pv_v, dg_v, vv_v, ct_v,
                     den, acc):
        c = lax.axis_index("c")
        s = lax.axis_index("s")
        base_w = (s * 2 + c) * per_w
        io16 = lax.iota(i32, 16)
        h8 = io16 & 7
        hi1 = jnp.where(io16 >= 8, 1, 0).astype(i32)

        rows = pl.ds(s * RPT, RPT)
        pltpu.sync_copy(z8_hbm, den.at[rows])
        pltpu.sync_copy(z8_hbm, acc.at[rows])
        plsc.subcore_barrier()

        base3 = s * p3w

        @pl.loop(0, n3)
        def _(ch):
            base = base3 + ch * c3
            pltpu.sync_copy(rs_hbm.at[pl.ds(base, c3)], i3_v)
            pltpu.sync_copy(p_hbm.at[pl.ds(base, c3)], p3_v)
            pltpu.sync_copy(p3_v, den.at[i3_v], add=True)

        plsc.subcore_barrier()
        pltpu.sync_copy(den.at[rows], dh_hbm.at[c, rows])
        plsc.subcore_barrier()

        @pl.loop(0, n4)
        def _(ch):
            base = base_w + ch * cb
            pltpu.sync_copy(rs_hbm.at[pl.ds(base, cb)], ir_v)
            pltpu.sync_copy(col_hbm.at[pl.ds(base, cb)], ic_v)
            pltpu.sync_copy(p_hbm.at[pl.ds(base, cb)], pv_v)
            pltpu.sync_copy(v_hbm.at[ic_v], vv_v)
            pltpu.sync_copy(dh_hbm.at[c].at[ir_v], dg_v)

            @plsc.parallel_loop(0, cb // 2, unroll=8)
            def _(ee):
                e2 = jnp.full((16,), 2 * ee, i32) + hi1
                pvx = plsc.load_gather(pv_v, [e2, h8])
                dgx = plsc.load_gather(dg_v, [e2, h8])
                vvx = plsc.load_gather(vv_v, [e2, h8])
                ctx = vvx * (pvx / (dgx + 1e-16))
                plsc.store_scatter(ct_v, [e2, h8], ctx)

            pltpu.sync_copy(ct_v, acc.at[ir_v], add=True)

        plsc.subcore_barrier()
        pltpu.sync_copy(acc.at[rows], out_hbm.at[c, rows])

    return aggr1_kernel(p1, v1t, row_s, col, z8)


def _tc_qkv(x, Wq, bq, Wk, bk, Wv):
    blk = 2000

    def body(x_ref, wq_ref, bq_ref, wk_ref, bk_ref, wv_ref, q_ref, k_ref, v_ref):
        xb = x_ref[...]
        q_ref[...] = jax.nn.relu(
            jnp.dot(xb, wq_ref[...], preferred_element_type=jnp.float32,
                    precision=_HIGH) + bq_ref[...])
        k_ref[...] = jax.nn.relu(
            jnp.dot(xb, wk_ref[...], preferred_element_type=jnp.float32,
                    precision=_HIGH) + bk_ref[...])
        v_ref[...] = jnp.dot(xb, wv_ref[...], preferred_element_type=jnp.float32,
                             precision=_HIGH)

    full = lambda i: (0, 0)
    o64 = jax.ShapeDtypeStruct((N, 64), jnp.float32)
    return pl.pallas_call(
        body,
        grid=(N // blk,),
        in_specs=[
            pl.BlockSpec((blk, DF), lambda i: (i, 0)),
            pl.BlockSpec((DF, 64), full),
            pl.BlockSpec((1, 64), full),
            pl.BlockSpec((DF, 64), full),
            pl.BlockSpec((1, 64), full),
            pl.BlockSpec((DF, 64), full),
        ],
        out_specs=[pl.BlockSpec((blk, 64), lambda i: (i, 0))] * 3,
        out_shape=[o64, o64, o64],
    )(x, Wq, bq.reshape(1, 64), Wk, bk.reshape(1, 64), Wv)


def _tc_layer1_tables(oa, ob, b0, Wq1b, bq1b, Wk1b, bk1b, W1p):
    blk = 2000

    def body(a_ref, b_ref, b0_ref, wq_ref, bq_ref, wk_ref, bk_ref, wv_ref,
             q_ref, k_ref, v_ref):
        h = jax.nn.relu(a_ref[...] + b_ref[...] + b0_ref[...])
        q_ref[...] = jax.nn.relu(
            jnp.dot(h, wq_ref[...], preferred_element_type=jnp.float32,
                    precision=_HIGH) + bq_ref[...])
        k_ref[...] = jax.nn.relu(
            jnp.dot(h, wk_ref[...], preferred_element_type=jnp.float32,
                    precision=_HIGH) + bk_ref[...])
        v_ref[...] = jnp.dot(h, wv_ref[...], preferred_element_type=jnp.float32,
                             precision=_HIGH)

    full = lambda i: (0, 0)
    o8 = jax.ShapeDtypeStruct((N, 8), jnp.float32)
    return pl.pallas_call(
        body,
        grid=(N // blk,),
        in_specs=[
            pl.BlockSpec((blk, 64), lambda i: (i, 0)),
            pl.BlockSpec((blk, 64), lambda i: (i, 0)),
            pl.BlockSpec((1, 64), full),
            pl.BlockSpec((64, 8), full),
            pl.BlockSpec((1, 8), full),
            pl.BlockSpec((64, 8), full),
            pl.BlockSpec((1, 8), full),
            pl.BlockSpec((64, 8), full),
        ],
        out_specs=[pl.BlockSpec((blk, 8), lambda i: (i, 0))] * 3,
        out_shape=[o8, o8, o8],
    )(oa, ob, b0.reshape(1, 64), Wq1b, bq1b, Wk1b, bk1b, W1p)


def _tc_final(o1a, o1b, b1p):
    blk = 2000

    def body(a_ref, b_ref, bias_ref, o_ref):
        o_ref[...] = a_ref[...] + b_ref[...] + bias_ref[...]

    return pl.pallas_call(
        body,
        grid=(N // blk,),
        in_specs=[
            pl.BlockSpec((blk, 8), lambda i: (i, 0)),
            pl.BlockSpec((blk, 8), lambda i: (i, 0)),
            pl.BlockSpec((1, 8), lambda i: (0, 0)),
        ],
        out_specs=pl.BlockSpec((blk, 8), lambda i: (i, 0)),
        out_shape=jax.ShapeDtypeStruct((N, 8), jnp.float32),
    )(o1a, o1b, b1p)


def kernel(x, edge_index, Wq0, bq0, Wk0, bk0, W0, b0, Wq1, bq1, Wk1, bk1, W1, b1):
    loops = jnp.arange(N, dtype=jnp.int32)
    pad = jnp.zeros((E_PAD - E_TOT,), jnp.int32)
    row = jnp.concatenate([edge_index[0].astype(jnp.int32), loops, pad])
    col = jnp.concatenate([edge_index[1].astype(jnp.int32), loops, pad])
    row_s = jnp.concatenate([edge_index[0].astype(jnp.int32), loops,
                             jnp.full((E_PAD - E_TOT,), N, jnp.int32)])

    z8 = jnp.zeros((RPT, 8), jnp.float32)
    z64 = jnp.zeros((RPT, 64), jnp.float32)

    perm = (jnp.arange(64) % 8) * 8 + jnp.arange(64) // 8
    Q, K, V = _tc_qkv(x, Wq0[:, perm], bq0[perm], Wk0[:, perm], bk0[perm], W0)
    p = _sc_attn0(Q, K, row, col)
    opart, _ = _sc_aggr0(p, V, row_s, col, z8, z64)

    Wq1b = jnp.broadcast_to(Wq1, (64, 8))
    bq1b = jnp.broadcast_to(bq1, (1, 8))
    Wk1b = jnp.broadcast_to(Wk1, (64, 8))
    bk1b = jnp.broadcast_to(bk1, (1, 8))
    W1p = jnp.pad(W1, ((0, 0), (0, 1)))
    q1t, k1t, v1t = _tc_layer1_tables(opart[0], opart[1], b0,
                                      Wq1b, bq1b, Wk1b, bk1b, W1p)
    p1 = _sc_attn1(q1t, k1t, row, col)
    o1part, _ = _sc_aggr1(p1, v1t, row_s, col, z8)
    out8 = _tc_final(o1part[0], o1part[1], jnp.pad(b1, (0, 1)).reshape(1, 8))
    return out8[:, :7]


# speedup vs baseline: 86.2750x; 1.5320x over previous
import jax
import jax.numpy as jnp
from jax import lax
from jax.experimental import pallas as pl
from jax.experimental.pallas import tpu as pltpu
from jax.experimental.pallas import tpu_sc as plsc

N = 10000
DF = 128
E_RAW = 320000
E_TOT = E_RAW + N
E_PAD = 331776
NW = 32
N_ACC = 10016
RPT = N_ACC // 16

_HIGH = lax.Precision.HIGHEST


def _mesh():
    return plsc.VectorSubcoreMesh(core_axis_name="c", subcore_axis_name="s")


_SC_PARAMS = pltpu.CompilerParams(use_tc_tiling_on_sc=False,
                                  needs_layout_passes=False)


def _vperm(v, idx):
    return lax.gather(
        v, idx.reshape(16, 1),
        lax.GatherDimensionNumbers(offset_dims=(), collapsed_slice_dims=(0,),
                                   start_index_map=(0,)),
        (1,), mode=lax.GatherScatterMode.PROMISE_IN_BOUNDS)


def _sc_attn0(q, k, row, col):
    ca = 288
    per_w = E_PAD // NW
    n_ch = per_w // ca

    i32 = jnp.int32
    f32 = jnp.float32
    scr = ([pltpu.VMEM((ca,), i32)] * 4 +
           [pltpu.VMEM((ca, 64), f32)] * 4 +
           [pltpu.VMEM((ca, 8), f32)] * 2 +
           [pltpu.SemaphoreType.DMA] * 6)

    @pl.kernel(out_type=jax.ShapeDtypeStruct((E_PAD, 8), f32),
               mesh=_mesh(), compiler_params=_SC_PARAMS, scratch_types=scr)
    def attn_kernel(q_hbm, k_hbm, row_hbm, col_hbm, p_hbm,
                    ir0, ir1, ic0, ic1, qe0, qe1, ke0, ke1, pv0, pv1,
                    sq0, sq1, sk0, sk1, wp0, wp1):
        c = lax.axis_index("c")
        s = lax.axis_index("s")
        base_w = (s * 2 + c) * per_w
        IR = (ir0, ir1)
        IC = (ic0, ic1)
        QE = (qe0, qe1)
        KE = (ke0, ke1)
        PV = (pv0, pv1)
        SQ = (sq0, sq1)
        SK = (sk0, sk1)
        WP = (wp0, wp1)
        io16 = lax.iota(i32, 16)
        p8 = io16 ^ 8
        h8 = io16 & 7
        lo8 = io16 < 8

        def issue(b, ch):
            base = base_w + ch * ca
            pltpu.sync_copy(row_hbm.at[pl.ds(base, ca)], IR[b])
            pltpu.sync_copy(col_hbm.at[pl.ds(base, ca)], IC[b])
            pltpu.async_copy(q_hbm.at[IR[b]], QE[b], SQ[b])
            pltpu.async_copy(k_hbm.at[IC[b]], KE[b], SK[b])

        for b in range(2):
            issue(b, b)

        @pl.loop(0, n_ch // 2)
        def _(gg):
            for b in range(2):
                ch = gg * 2 + b
                pltpu.make_async_copy(q_hbm.at[IR[b]], QE[b], SQ[b]).wait()
                pltpu.make_async_copy(k_hbm.at[IC[b]], KE[b], SK[b]).wait()

                @pl.when(gg > 0)
                def _():
                    pltpu.make_async_copy(PV[b], p_hbm.at[pl.ds(0, ca)],
                                          WP[b]).wait()

                qe_v, ke_v, pv_v = QE[b], KE[b], PV[b]

                @plsc.parallel_loop(0, ca, unroll=4)
                def _(e):
                    acc = None
                    for j in range(4):
                        qv = qe_v[e, pl.ds(j * 16, 16)]
                        kv = ke_v[e, pl.ds(j * 16, 16)]
                        pr = qv * kv
                        acc = pr if acc is None else acc + pr
                    att = acc + _vperm(acc, p8)
                    t = jnp.exp(att)
                    plsc.store_scatter(pv_v, [jnp.full((16,), e, i32), h8],
                                       t, mask=lo8)

                pltpu.async_copy(PV[b], p_hbm.at[pl.ds(base_w + ch * ca, ca)],
                                 WP[b])

                @pl.when(ch + 2 < n_ch)
                def _():
                    issue(b, ch + 2)

        for b in range(2):
            pltpu.make_async_copy(PV[b], p_hbm.at[pl.ds(0, ca)], WP[b]).wait()

    return attn_kernel(q, k, row, col)


def _sc_aggr0(p, v, row_s, col, z8, z64):
    cb = 216
    per_w = E_PAD // NW
    n4 = per_w // cb
    c3 = 648
    p3w = E_PAD // 16
    n3 = p3w // c3

    i32 = jnp.int32
    f32 = jnp.float32
    scr = ([pltpu.VMEM((c3,), i32)] * 2 + [pltpu.VMEM((c3, 8), f32)] * 2 +
           [pltpu.VMEM((cb,), i32)] * 4 +
           [pltpu.VMEM((cb, 8), f32)] * 4 + [pltpu.VMEM((cb, 64), f32)] * 4 +
           [pltpu.VMEM_SHARED((N_ACC, 8), f32),
            pltpu.VMEM_SHARED((N_ACC, 64), f32)] +
           [pltpu.SemaphoreType.DMA] * 8)

    @pl.kernel(out_type=(jax.ShapeDtypeStruct((2, N_ACC, 64), f32),
                         jax.ShapeDtypeStruct((2, N_ACC, 8), f32)),
               mesh=_mesh(), compiler_params=_SC_PARAMS, scratch_types=scr)
    def aggr_kernel(p_hbm, v_hbm, rs_hbm, col_hbm, z8_hbm, z64_hbm,
                    out_hbm, dh_hbm,
                    i30, i31, p30, p31, ir0, ir1, ic0, ic1,
                    pv0, pv1, dg0, dg1, vv0, vv1, ct0, ct1,
                    den, acc, s30, s31, sp0, sp1, sv0, sv1, sd0, sd1):
        c = lax.axis_index("c")
        s = lax.axis_index("s")
        base_w = (s * 2 + c) * per_w
        I3 = (i30, i31)
        P3 = (p30, p31)
        IR = (ir0, ir1)
        IC = (ic0, ic1)
        PV = (pv0, pv1)
        DG = (dg0, dg1)
        VV = (vv0, vv1)
        CT = (ct0, ct1)
        S3 = (s30, s31)
        SP = (sp0, sp1)
        SV = (sv0, sv1)
        SD = (sd0, sd1)
        io16 = lax.iota(i32, 16)
        hi1 = jnp.where(io16 >= 8, 1, 0).astype(i32)

        rows = pl.ds(s * RPT, RPT)
        pltpu.sync_copy(z8_hbm, den.at[rows])
        pltpu.sync_copy(z64_hbm, acc.at[rows])
        plsc.subcore_barrier()

        base3 = s * p3w

        def issue3(b, ch):
            base = base3 + ch * c3
            pltpu.sync_copy(rs_hbm.at[pl.ds(base, c3)], I3[b])
            pltpu.async_copy(p_hbm.at[pl.ds(base, c3)], P3[b], S3[b])

        for b in range(2):
            issue3(b, b)

        @pl.loop(0, n3 // 2)
        def _(gg):
            for b in range(2):
                ch = gg * 2 + b
                pltpu.make_async_copy(p_hbm.at[pl.ds(0, c3)], P3[b],
                                      S3[b]).wait()
                pltpu.sync_copy(P3[b], den.at[I3[b]], add=True)

                @pl.when(ch + 2 < n3)
                def _():
                    issue3(b, ch + 2)

        plsc.subcore_barrier()
        pltpu.sync_copy(den.at[rows], dh_hbm.at[c, rows])
        plsc.subcore_barrier()

        def issue4(b, ch):
            base = base_w + ch * cb
            pltpu.sync_copy(rs_hbm.at[pl.ds(base, cb)], IR[b])
            pltpu.sync_copy(col_hbm.at[pl.ds(base, cb)], IC[b])
            pltpu.async_copy(p_hbm.at[pl.ds(base, cb)], PV[b], SP[b])
            pltpu.async_copy(v_hbm.at[IC[b]], VV[b], SV[b])
            pltpu.async_copy(dh_hbm.at[c].at[IR[b]], DG[b], SD[b])

        for b in range(2):
            issue4(b, b)

        @pl.loop(0, n4 // 2)
        def _(gg):
            for b in range(2):
                ch = gg * 2 + b
                pltpu.make_async_copy(p_hbm.at[pl.ds(0, cb)], PV[b],
                                      SP[b]).wait()
                pltpu.make_async_copy(v_hbm.at[IC[b]], VV[b], SV[b]).wait()
                pltpu.make_async_copy(dh_hbm.at[c].at[IR[b]], DG[b],
                                      SD[b]).wait()

                pv_v, dg_v, vv_v, ct_v = PV[b], DG[b], VV[b], CT[b]

                @plsc.parallel_loop(0, cb, unroll=4)
                def _(e):
                    se = jnp.full((16,), e, i32)
                    pv8 = plsc.load_gather(pv_v, [se, io16 & 7])
                    dg8 = plsc.load_gather(dg_v, [se, io16 & 7])
                    av = pv8 / (dg8 + 1e-16)
                    for j in range(4):
                        avj = _vperm(av, jnp.full((16,), 2 * j, i32) + hi1)
                        sl = pl.ds(j * 16, 16)
                        ct_v[e, sl] = vv_v[e, sl] * avj

                pltpu.sync_copy(ct_v, acc.at[IR[b]], add=True)

                @pl.when(ch + 2 < n4)
                def _():
                    issue4(b, ch + 2)

        plsc.subcore_barrier()
        pltpu.sync_copy(acc.at[rows], out_hbm.at[c, rows])

    return aggr_kernel(p, v, row_s, col, z8, z64)


def _sc_attn1(q1t, k1t, row, col):
    ca = 648
    per_w = E_PAD // NW
    n_ch = per_w // ca

    i32 = jnp.int32
    f32 = jnp.float32
    scr = ([pltpu.VMEM((ca,), i32)] * 2 + [pltpu.VMEM((ca, 8), f32)] * 3)

    @pl.kernel(out_type=jax.ShapeDtypeStruct((E_PAD, 8), f32),
               mesh=_mesh(), compiler_params=_SC_PARAMS, scratch_types=scr)
    def attn1_kernel(q_hbm, k_hbm, row_hbm, col_hbm, p_hbm,
                     ir_v, ic_v, qe_v, ke_v, pv_v):
        c = lax.axis_index("c")
        s = lax.axis_index("s")
        base_w = (s * 2 + c) * per_w
        io16 = lax.iota(i32, 16)
        h8 = io16 & 7
        hi1 = jnp.where(io16 >= 8, 1, 0).astype(i32)

        @pl.loop(0, n_ch)
        def _(ch):
            base = base_w + ch * ca
            pltpu.sync_copy(row_hbm.at[pl.ds(base, ca)], ir_v)
            pltpu.sync_copy(col_hbm.at[pl.ds(base, ca)], ic_v)
            pltpu.sync_copy(q_hbm.at[ir_v], qe_v)
            pltpu.sync_copy(k_hbm.at[ic_v], ke_v)

            @plsc.parallel_loop(0, ca // 2, unroll=8)
            def _(ee):
                e2 = jnp.full((16,), 2 * ee, i32) + hi1
                qv = plsc.load_gather(qe_v, [e2, h8])
                kv = plsc.load_gather(ke_v, [e2, h8])
                plsc.store_scatter(pv_v, [e2, h8], jnp.exp(qv * kv))

            pltpu.sync_copy(pv_v, p_hbm.at[pl.ds(base, ca)])

    return attn1_kernel(q1t, k1t, row, col)


def _sc_aggr1(p1, v1t, row_s, col, z8):
    cb = 648
    per_w = E_PAD // NW
    n4 = per_w // cb
    c3 = 648
    p3w = E_PAD // 16
    n3 = p3w // c3

    i32 = jnp.int32
    f32 = jnp.float32
    scr = ([pltpu.VMEM((c3,), i32)] * 2 + [pltpu.VMEM((c3, 8), f32)] * 2 +
           [pltpu.VMEM((cb,), i32)] * 4 +
           [pltpu.VMEM((cb, 8), f32)] * 8 +
           [pltpu.VMEM_SHARED((N_ACC, 8), f32),
            pltpu.VMEM_SHARED((N_ACC, 8), f32)] +
           [pltpu.SemaphoreType.DMA] * 8)

    @pl.kernel(out_type=(jax.ShapeDtypeStruct((2, N_ACC, 8), f32),
                         jax.ShapeDtypeStruct((2, N_ACC, 8), f32)),
               mesh=_mesh(), compiler_params=_SC_PARAMS, scratch_types=scr)
    def aggr1_kernel(p_hbm, v_hbm, rs_hbm, col_hbm, z8_hbm,
                     out_hbm, dh_hbm,
                     i30, i31, p30, p31, ir0, ir1, ic0, ic1,
                     pv0, pv1, dg0, dg1, vv0, vv1, ct0, ct1,
                     den, acc, s30, s31, sp0, sp1, sv0, sv1, sd0, sd1):
        c = lax.axis_index("c")
        s = lax.axis_index("s")
        base_w = (s * 2 + c) * per_w
        I3 = (i30, i31)
        P3 = (p30, p31)
        IR = (ir0, ir1)
        IC = (ic0, ic1)
        PV = (pv0, pv1)
        DG = (dg0, dg1)
        VV = (vv0, vv1)
        CT = (ct0, ct1)
        S3 = (s30, s31)
        SP = (sp0, sp1)
        SV = (sv0, sv1)
        SD = (sd0, sd1)
        io16 = lax.iota(i32, 16)
        h8 = io16 & 7
        hi1 = jnp.where(io16 >= 8, 1, 0).astype(i32)

        rows = pl.ds(s * RPT, RPT)
        pltpu.sync_copy(z8_hbm, den.at[rows])
        pltpu.sync_copy(z8_hbm, acc.at[rows])
        plsc.subcore_barrier()

        base3 = s * p3w

        def issue3(b, ch):
            base = base3 + ch * c3
            pltpu.sync_copy(rs_hbm.at[pl.ds(base, c3)], I3[b])
            pltpu.async_copy(p_hbm.at[pl.ds(base, c3)], P3[b], S3[b])

        for b in range(2):
            issue3(b, b)

        @pl.loop(0, n3 // 2)
        def _(gg):
            for b in range(2):
                ch = gg * 2 + b
                pltpu.make_async_copy(p_hbm.at[pl.ds(0, c3)], P3[b],
                                      S3[b]).wait()
                pltpu.sync_copy(P3[b], den.at[I3[b]], add=True)

                @pl.when(ch + 2 < n3)
                def _():
                    issue3(b, ch + 2)

        plsc.subcore_barrier()
        pltpu.sync_copy(den.at[rows], dh_hbm.at[c, rows])
        plsc.subcore_barrier()

        def issue4(b, ch):
            base = base_w + ch * cb
            pltpu.sync_copy(rs_hbm.at[pl.ds(base, cb)], IR[b])
            pltpu.sync_copy(col_hbm.at[pl.ds(base, cb)], IC[b])
            pltpu.async_copy(p_hbm.at[pl.ds(base, cb)], PV[b], SP[b])
            pltpu.async_copy(v_hbm.at[IC[b]], VV[b], SV[b])
            pltpu.async_copy(dh_hbm.at[c].at[IR[b]], DG[b], SD[b])

        for b in range(2):
            issue4(b, b)

        @pl.loop(0, n4 // 2)
        def _(gg):
            for b in range(2):
                ch = gg * 2 + b
                pltpu.make_async_copy(p_hbm.at[pl.ds(0, cb)], PV[b],
                                      SP[b]).wait()
                pltpu.make_async_copy(v_hbm.at[IC[b]], VV[b], SV[b]).wait()
                pltpu.make_async_copy(dh_hbm.at[c].at[IR[b]], DG[b],
                                      SD[b]).wait()

                pv_v, dg_v, vv_v, ct_v = PV[b], DG[b], VV[b], CT[b]

                @plsc.parallel_loop(0, cb // 2, unroll=8)
                def _(ee):
                    e2 = jnp.full((16,), 2 * ee, i32) + hi1
                    pvx = plsc.load_gather(pv_v, [e2, h8])
                    dgx = plsc.load_gather(dg_v, [e2, h8])
                    vvx = plsc.load_gather(vv_v, [e2, h8])
                    ctx = vvx * (pvx / (dgx + 1e-16))
                    plsc.store_scatter(ct_v, [e2, h8], ctx)

                pltpu.sync_copy(ct_v, acc.at[IR[b]], add=True)

                @pl.when(ch + 2 < n4)
                def _():
                    issue4(b, ch + 2)

        plsc.subcore_barrier()
        pltpu.sync_copy(acc.at[rows], out_hbm.at[c, rows])

    return aggr1_kernel(p1, v1t, row_s, col, z8)


def _tc_qkv(x, Wq, bq, Wk, bk, Wv):
    blk = 2000

    def body(x_ref, wq_ref, bq_ref, wk_ref, bk_ref, wv_ref, q_ref, k_ref, v_ref):
        xb = x_ref[...]
        q_ref[...] = jax.nn.relu(
            jnp.dot(xb, wq_ref[...], preferred_element_type=jnp.float32,
                    precision=_HIGH) + bq_ref[...])
        k_ref[...] = jax.nn.relu(
            jnp.dot(xb, wk_ref[...], preferred_element_type=jnp.float32,
                    precision=_HIGH) + bk_ref[...])
        v_ref[...] = jnp.dot(xb, wv_ref[...], preferred_element_type=jnp.float32,
                             precision=_HIGH)

    full = lambda i: (0, 0)
    o64 = jax.ShapeDtypeStruct((N, 64), jnp.float32)
    return pl.pallas_call(
        body,
        grid=(N // blk,),
        in_specs=[
            pl.BlockSpec((blk, DF), lambda i: (i, 0)),
            pl.BlockSpec((DF, 64), full),
            pl.BlockSpec((1, 64), full),
            pl.BlockSpec((DF, 64), full),
            pl.BlockSpec((1, 64), full),
            pl.BlockSpec((DF, 64), full),
        ],
        out_specs=[pl.BlockSpec((blk, 64), lambda i: (i, 0))] * 3,
        out_shape=[o64, o64, o64],
    )(x, Wq, bq.reshape(1, 64), Wk, bk.reshape(1, 64), Wv)


def _tc_layer1_tables(oa, ob, b0, Wq1b, bq1b, Wk1b, bk1b, W1p):
    blk = 2000

    def body(a_ref, b_ref, b0_ref, wq_ref, bq_ref, wk_ref, bk_ref, wv_ref,
             q_ref, k_ref, v_ref):
        h = jax.nn.relu(a_ref[...] + b_ref[...] + b0_ref[...])
        q_ref[...] = jax.nn.relu(
            jnp.dot(h, wq_ref[...], preferred_element_type=jnp.float32,
                    precision=_HIGH) + bq_ref[...])
        k_ref[...] = jax.nn.relu(
            jnp.dot(h, wk_ref[...], preferred_element_type=jnp.float32,
                    precision=_HIGH) + bk_ref[...])
        v_ref[...] = jnp.dot(h, wv_ref[...], preferred_element_type=jnp.float32,
                             precision=_HIGH)

    full = lambda i: (0, 0)
    o8 = jax.ShapeDtypeStruct((N, 8), jnp.float32)
    return pl.pallas_call(
        body,
        grid=(N // blk,),
        in_specs=[
            pl.BlockSpec((blk, 64), lambda i: (i, 0)),
            pl.BlockSpec((blk, 64), lambda i: (i, 0)),
            pl.BlockSpec((1, 64), full),
            pl.BlockSpec((64, 8), full),
            pl.BlockSpec((1, 8), full),
            pl.BlockSpec((64, 8), full),
            pl.BlockSpec((1, 8), full),
            pl.BlockSpec((64, 8), full),
        ],
        out_specs=[pl.BlockSpec((blk, 8), lambda i: (i, 0))] * 3,
        out_shape=[o8, o8, o8],
    )(oa, ob, b0.reshape(1, 64), Wq1b, bq1b, Wk1b, bk1b, W1p)


def _tc_final(o1a, o1b, b1p):
    blk = 2000

    def body(a_ref, b_ref, bias_ref, o_ref):
        o_ref[...] = a_ref[...] + b_ref[...] + bias_ref[...]

    return pl.pallas_call(
        body,
        grid=(N // blk,),
        in_specs=[
            pl.BlockSpec((blk, 8), lambda i: (i, 0)),
            pl.BlockSpec((blk, 8), lambda i: (i, 0)),
            pl.BlockSpec((1, 8), lambda i: (0, 0)),
        ],
        out_specs=pl.BlockSpec((blk, 8), lambda i: (i, 0)),
        out_shape=jax.ShapeDtypeStruct((N, 8), jnp.float32),
    )(o1a, o1b, b1p)


def kernel(x, edge_index, Wq0, bq0, Wk0, bk0, W0, b0, Wq1, bq1, Wk1, bk1, W1, b1):
    loops = jnp.arange(N, dtype=jnp.int32)
    pad = jnp.zeros((E_PAD - E_TOT,), jnp.int32)
    row = jnp.concatenate([edge_index[0].astype(jnp.int32), loops, pad])
    col = jnp.concatenate([edge_index[1].astype(jnp.int32), loops, pad])
    row_s = jnp.concatenate([edge_index[0].astype(jnp.int32), loops,
                             jnp.full((E_PAD - E_TOT,), N, jnp.int32)])

    z8 = jnp.zeros((RPT, 8), jnp.float32)
    z64 = jnp.zeros((RPT, 64), jnp.float32)

    perm = (jnp.arange(64) % 8) * 8 + jnp.arange(64) // 8
    Q, K, V = _tc_qkv(x, Wq0[:, perm], bq0[perm], Wk0[:, perm], bk0[perm], W0)
    p = _sc_attn0(Q, K, row, col)
    opart, _ = _sc_aggr0(p, V, row_s, col, z8, z64)

    Wq1b = jnp.broadcast_to(Wq1, (64, 8))
    bq1b = jnp.broadcast_to(bq1, (1, 8))
    Wk1b = jnp.broadcast_to(Wk1, (64, 8))
    bk1b = jnp.broadcast_to(bk1, (1, 8))
    W1p = jnp.pad(W1, ((0, 0), (0, 1)))
    q1t, k1t, v1t = _tc_layer1_tables(opart[0], opart[1], b0,
                                      Wq1b, bq1b, Wk1b, bk1b, W1p)
    p1 = _sc_attn1(q1t, k1t, row, col)
    o1part, _ = _sc_aggr1(p1, v1t, row_s, col, z8)
    out8 = _tc_final(o1part[0], o1part[1], jnp.pad(b1, (0, 1)).reshape(1, 8))
    return out8[:, :7]


# speedup vs baseline: 92.7137x; 1.0746x over previous
import jax
import jax.numpy as jnp
from jax import lax
from jax.experimental import pallas as pl
from jax.experimental.pallas import tpu as pltpu
from jax.experimental.pallas import tpu_sc as plsc

N = 10000
DF = 128
E_RAW = 320000
E_TOT = E_RAW + N
E_PAD = 331776
NW = 32
N_ACC = 10016
RPT = N_ACC // 16

_HIGH = lax.Precision.HIGHEST


def _mesh():
    return plsc.VectorSubcoreMesh(core_axis_name="c", subcore_axis_name="s")


_SC_PARAMS = pltpu.CompilerParams(use_tc_tiling_on_sc=False,
                                  needs_layout_passes=False)


def _vperm(v, idx):
    return lax.gather(
        v, idx.reshape(16, 1),
        lax.GatherDimensionNumbers(offset_dims=(), collapsed_slice_dims=(0,),
                                   start_index_map=(0,)),
        (1,), mode=lax.GatherScatterMode.PROMISE_IN_BOUNDS)


def _sc_attn0(q, k, row, row_s, col, z8):
    ca = 288
    per_w = E_PAD // NW
    n_ch = per_w // ca

    i32 = jnp.int32
    f32 = jnp.float32
    scr = ([pltpu.VMEM((ca,), i32)] * 6 +
           [pltpu.VMEM((ca, 64), f32)] * 4 +
           [pltpu.VMEM((ca, 8), f32)] * 2 +
           [pltpu.VMEM_SHARED((N_ACC, 8), f32)] +
           [pltpu.SemaphoreType.DMA] * 6)

    @pl.kernel(out_type=(jax.ShapeDtypeStruct((E_PAD, 8), f32),
                         jax.ShapeDtypeStruct((2, N_ACC, 8), f32)),
               mesh=_mesh(), compiler_params=_SC_PARAMS, scratch_types=scr)
    def attn_kernel(q_hbm, k_hbm, row_hbm, rs_hbm, col_hbm, z8_hbm,
                    p_hbm, dh_hbm,
                    ir0, ir1, is0, is1, ic0, ic1, qe0, qe1, ke0, ke1,
                    pv0, pv1, den,
                    sq0, sq1, sk0, sk1, wp0, wp1):
        c = lax.axis_index("c")
        s = lax.axis_index("s")
        base_w = (s * 2 + c) * per_w
        IR = (ir0, ir1)
        IS = (is0, is1)
        IC = (ic0, ic1)
        QE = (qe0, qe1)
        KE = (ke0, ke1)
        PV = (pv0, pv1)
        SQ = (sq0, sq1)
        SK = (sk0, sk1)
        WP = (wp0, wp1)
        io16 = lax.iota(i32, 16)
        p8 = io16 ^ 8
        h8 = io16 & 7
        lo8 = io16 < 8

        rows = pl.ds(s * RPT, RPT)
        pltpu.sync_copy(z8_hbm, den.at[rows])
        plsc.subcore_barrier()

        def issue(b, ch):
            base = base_w + ch * ca
            pltpu.sync_copy(row_hbm.at[pl.ds(base, ca)], IR[b])
            pltpu.sync_copy(rs_hbm.at[pl.ds(base, ca)], IS[b])
            pltpu.sync_copy(col_hbm.at[pl.ds(base, ca)], IC[b])
            pltpu.async_copy(q_hbm.at[IR[b]], QE[b], SQ[b])
            pltpu.async_copy(k_hbm.at[IC[b]], KE[b], SK[b])

        for b in range(2):
            issue(b, b)

        @pl.loop(0, n_ch // 2)
        def _(gg):
            for b in range(2):
                ch = gg * 2 + b
                pltpu.make_async_copy(q_hbm.at[IR[b]], QE[b], SQ[b]).wait()
                pltpu.make_async_copy(k_hbm.at[IC[b]], KE[b], SK[b]).wait()

                @pl.when(gg > 0)
                def _():
                    pltpu.make_async_copy(PV[b], p_hbm.at[pl.ds(0, ca)],
                                          WP[b]).wait()

                qe_v, ke_v, pv_v = QE[b], KE[b], PV[b]

                @plsc.parallel_loop(0, ca, unroll=4)
                def _(e):
                    acc = None
                    for j in range(4):
                        qv = qe_v[e, pl.ds(j * 16, 16)]
                        kv = ke_v[e, pl.ds(j * 16, 16)]
                        pr = qv * kv
                        acc = pr if acc is None else acc + pr
                    att = acc + _vperm(acc, p8)
                    t = jnp.exp(att)
                    plsc.store_scatter(pv_v, [jnp.full((16,), e, i32), h8],
                                       t, mask=lo8)

                pltpu.sync_copy(PV[b], den.at[IS[b]], add=True)
                pltpu.async_copy(PV[b], p_hbm.at[pl.ds(base_w + ch * ca, ca)],
                                 WP[b])

                @pl.when(ch + 2 < n_ch)
                def _():
                    issue(b, ch + 2)

        for b in range(2):
            pltpu.make_async_copy(PV[b], p_hbm.at[pl.ds(0, ca)], WP[b]).wait()
        plsc.subcore_barrier()
        pltpu.sync_copy(den.at[rows], dh_hbm.at[c, rows])

    return attn_kernel(q, k, row, row_s, col, z8)


def _sc_aggr0(p, v, row_s, col, z64):
    cb = 216
    per_w = E_PAD // NW
    n4 = per_w // cb

    i32 = jnp.int32
    f32 = jnp.float32
    scr = ([pltpu.VMEM((cb,), i32)] * 4 +
           [pltpu.VMEM((cb, 8), f32)] * 2 + [pltpu.VMEM((cb, 64), f32)] * 4 +
           [pltpu.VMEM_SHARED((N_ACC, 64), f32)] +
           [pltpu.SemaphoreType.DMA] * 4)

    @pl.kernel(out_type=jax.ShapeDtypeStruct((2, N_ACC, 64), f32),
               mesh=_mesh(), compiler_params=_SC_PARAMS, scratch_types=scr)
    def aggr_kernel(p_hbm, v_hbm, rs_hbm, col_hbm, z64_hbm, out_hbm,
                    ir0, ir1, ic0, ic1, pv0, pv1, vv0, vv1, ct0, ct1,
                    acc, sp0, sp1, sv0, sv1):
        c = lax.axis_index("c")
        s = lax.axis_index("s")
        base_w = (s * 2 + c) * per_w
        IR = (ir0, ir1)
        IC = (ic0, ic1)
        PV = (pv0, pv1)
        VV = (vv0, vv1)
        CT = (ct0, ct1)
        SP = (sp0, sp1)
        SV = (sv0, sv1)
        io16 = lax.iota(i32, 16)
        hi1 = jnp.where(io16 >= 8, 1, 0).astype(i32)

        rows = pl.ds(s * RPT, RPT)
        pltpu.sync_copy(z64_hbm, acc.at[rows])
        plsc.subcore_barrier()

        def issue4(b, ch):
            base = base_w + ch * cb
            pltpu.sync_copy(rs_hbm.at[pl.ds(base, cb)], IR[b])
            pltpu.sync_copy(col_hbm.at[pl.ds(base, cb)], IC[b])
            pltpu.async_copy(p_hbm.at[pl.ds(base, cb)], PV[b], SP[b])
            pltpu.async_copy(v_hbm.at[IC[b]], VV[b], SV[b])

        for b in range(2):
            issue4(b, b)

        @pl.loop(0, n4 // 2)
        def _(gg):
            for b in range(2):
                ch = gg * 2 + b
                pltpu.make_async_copy(p_hbm.at[pl.ds(0, cb)], PV[b],
                                      SP[b]).wait()
                pltpu.make_async_copy(v_hbm.at[IC[b]], VV[b], SV[b]).wait()

                pv_v, vv_v, ct_v = PV[b], VV[b], CT[b]

                @plsc.parallel_loop(0, cb, unroll=4)
                def _(e):
                    se = jnp.full((16,), e, i32)
                    pv8 = plsc.load_gather(pv_v, [se, io16 & 7])
                    for j in range(4):
                        pvj = _vperm(pv8, jnp.full((16,), 2 * j, i32) + hi1)
                        sl = pl.ds(j * 16, 16)
                        ct_v[e, sl] = vv_v[e, sl] * pvj

                pltpu.sync_copy(ct_v, acc.at[IR[b]], add=True)

                @pl.when(ch + 2 < n4)
                def _():
                    issue4(b, ch + 2)

        plsc.subcore_barrier()
        pltpu.sync_copy(acc.at[rows], out_hbm.at[c, rows])

    return aggr_kernel(p, v, row_s, col, z64)


def _sc_attn1(q1t, k1t, row, row_s, col, z8):
    ca = 648
    per_w = E_PAD // NW
    n_ch = per_w // ca

    i32 = jnp.int32
    f32 = jnp.float32
    scr = ([pltpu.VMEM((ca,), i32)] * 6 + [pltpu.VMEM((ca, 8), f32)] * 6 +
           [pltpu.VMEM_SHARED((N_ACC, 8), f32)] +
           [pltpu.SemaphoreType.DMA] * 6)

    @pl.kernel(out_type=(jax.ShapeDtypeStruct((E_PAD, 8), f32),
                         jax.ShapeDtypeStruct((2, N_ACC, 8), f32)),
               mesh=_mesh(), compiler_params=_SC_PARAMS, scratch_types=scr)
    def attn1_kernel(q_hbm, k_hbm, row_hbm, rs_hbm, col_hbm, z8_hbm,
                     p_hbm, dh_hbm,
                     ir0, ir1, is0, is1, ic0, ic1,
                     qe0, qe1, ke0, ke1, pv0, pv1, den,
                     sq0, sq1, sk0, sk1, wp0, wp1):
        c = lax.axis_index("c")
        s = lax.axis_index("s")
        base_w = (s * 2 + c) * per_w
        IR = (ir0, ir1)
        IS = (is0, is1)
        IC = (ic0, ic1)
        QE = (qe0, qe1)
        KE = (ke0, ke1)
        PV = (pv0, pv1)
        SQ = (sq0, sq1)
        SK = (sk0, sk1)
        WP = (wp0, wp1)
        io16 = lax.iota(i32, 16)
        h8 = io16 & 7
        hi1 = jnp.where(io16 >= 8, 1, 0).astype(i32)

        rows = pl.ds(s * RPT, RPT)
        pltpu.sync_copy(z8_hbm, den.at[rows])
        plsc.subcore_barrier()

        def issue(b, ch):
            base = base_w + ch * ca
            pltpu.sync_copy(row_hbm.at[pl.ds(base, ca)], IR[b])
            pltpu.sync_copy(rs_hbm.at[pl.ds(base, ca)], IS[b])
            pltpu.sync_copy(col_hbm.at[pl.ds(base, ca)], IC[b])
            pltpu.async_copy(q_hbm.at[IR[b]], QE[b], SQ[b])
            pltpu.async_copy(k_hbm.at[IC[b]], KE[b], SK[b])

        for b in range(2):
            issue(b, b)

        @pl.loop(0, n_ch // 2)
        def _(gg):
            for b in range(2):
                ch = gg * 2 + b
                pltpu.make_async_copy(q_hbm.at[IR[b]], QE[b], SQ[b]).wait()
                pltpu.make_async_copy(k_hbm.at[IC[b]], KE[b], SK[b]).wait()

                @pl.when(gg > 0)
                def _():
                    pltpu.make_async_copy(PV[b], p_hbm.at[pl.ds(0, ca)],
                                          WP[b]).wait()

                qe_v, ke_v, pv_v = QE[b], KE[b], PV[b]

                @plsc.parallel_loop(0, ca // 2, unroll=8)
                def _(ee):
                    e2 = jnp.full((16,), 2 * ee, i32) + hi1
                    qv = plsc.load_gather(qe_v, [e2, h8])
                    kv = plsc.load_gather(ke_v, [e2, h8])
                    plsc.store_scatter(pv_v, [e2, h8], jnp.exp(qv * kv))

                pltpu.sync_copy(pv_v, den.at[IS[b]], add=True)
                pltpu.async_copy(pv_v, p_hbm.at[pl.ds(base_w + ch * ca, ca)],
                                 WP[b])

                @pl.when(ch + 2 < n_ch)
                def _():
                    issue(b, ch + 2)

        for b in range(2):
            pltpu.make_async_copy(PV[b], p_hbm.at[pl.ds(0, ca)], WP[b]).wait()
        plsc.subcore_barrier()
        pltpu.sync_copy(den.at[rows], dh_hbm.at[c, rows])

    return attn1_kernel(q1t, k1t, row, row_s, col, z8)


def _sc_aggr1(p1, v1t, row_s, col, z8):
    cb = 648
    per_w = E_PAD // NW
    n4 = per_w // cb

    i32 = jnp.int32
    f32 = jnp.float32
    scr = ([pltpu.VMEM((cb,), i32)] * 4 +
           [pltpu.VMEM((cb, 8), f32)] * 6 +
           [pltpu.VMEM_SHARED((N_ACC, 8), f32)] +
           [pltpu.SemaphoreType.DMA] * 4)

    @pl.kernel(out_type=jax.ShapeDtypeStruct((2, N_ACC, 8), f32),
               mesh=_mesh(), compiler_params=_SC_PARAMS, scratch_types=scr)
    def aggr1_kernel(p_hbm, v_hbm, rs_hbm, col_hbm, z8_hbm, out_hbm,
                     ir0, ir1, ic0, ic1,
                     pv0, pv1, vv0, vv1, ct0, ct1,
                     acc, sp0, sp1, sv0, sv1):
        c = lax.axis_index("c")
        s = lax.axis_index("s")
        base_w = (s * 2 + c) * per_w
        IR = (ir0, ir1)
        IC = (ic0, ic1)
        PV = (pv0, pv1)
        VV = (vv0, vv1)
        CT = (ct0, ct1)
        SP = (sp0, sp1)
        SV = (sv0, sv1)
        io16 = lax.iota(i32, 16)
        h8 = io16 & 7
        hi1 = jnp.where(io16 >= 8, 1, 0).astype(i32)

        rows = pl.ds(s * RPT, RPT)
        pltpu.sync_copy(z8_hbm, acc.at[rows])
        plsc.subcore_barrier()

        def issue4(b, ch):
            base = base_w + ch * cb
            pltpu.sync_copy(rs_hbm.at[pl.ds(base, cb)], IR[b])
            pltpu.sync_copy(col_hbm.at[pl.ds(base, cb)], IC[b])
            pltpu.async_copy(p_hbm.at[pl.ds(base, cb)], PV[b], SP[b])
            pltpu.async_copy(v_hbm.at[IC[b]], VV[b], SV[b])

        for b in range(2):
            issue4(b, b)

        @pl.loop(0, n4 // 2)
        def _(gg):
            for b in range(2):
                ch = gg * 2 + b
                pltpu.make_async_copy(p_hbm.at[pl.ds(0, cb)], PV[b],
                                      SP[b]).wait()
                pltpu.make_async_copy(v_hbm.at[IC[b]], VV[b], SV[b]).wait()

                pv_v, vv_v, ct_v = PV[b], VV[b], CT[b]

                @plsc.parallel_loop(0, cb // 2, unroll=8)
                def _(ee):
                    e2 = jnp.full((16,), 2 * ee, i32) + hi1
                    pvx = plsc.load_gather(pv_v, [e2, h8])
                    vvx = plsc.load_gather(vv_v, [e2, h8])
                    plsc.store_scatter(ct_v, [e2, h8], vvx * pvx)

                pltpu.sync_copy(ct_v, acc.at[IR[b]], add=True)

                @pl.when(ch + 2 < n4)
                def _():
                    issue4(b, ch + 2)

        plsc.subcore_barrier()
        pltpu.sync_copy(acc.at[rows], out_hbm.at[c, rows])

    return aggr1_kernel(p1, v1t, row_s, col, z8)


def _tc_qkv(x, Wq, bq, Wk, bk, Wv):
    blk = 2000

    def body(x_ref, wq_ref, bq_ref, wk_ref, bk_ref, wv_ref, q_ref, k_ref, v_ref):
        xb = x_ref[...]
        q_ref[...] = jax.nn.relu(
            jnp.dot(xb, wq_ref[...], preferred_element_type=jnp.float32,
                    precision=_HIGH) + bq_ref[...])
        k_ref[...] = jax.nn.relu(
            jnp.dot(xb, wk_ref[...], preferred_element_type=jnp.float32,
                    precision=_HIGH) + bk_ref[...])
        v_ref[...] = jnp.dot(xb, wv_ref[...], preferred_element_type=jnp.float32,
                             precision=_HIGH)

    full = lambda i: (0, 0)
    o64 = jax.ShapeDtypeStruct((N, 64), jnp.float32)
    return pl.pallas_call(
        body,
        grid=(N // blk,),
        in_specs=[
            pl.BlockSpec((blk, DF), lambda i: (i, 0)),
            pl.BlockSpec((DF, 64), full),
            pl.BlockSpec((1, 64), full),
            pl.BlockSpec((DF, 64), full),
            pl.BlockSpec((1, 64), full),
            pl.BlockSpec((DF, 64), full),
        ],
        out_specs=[pl.BlockSpec((blk, 64), lambda i: (i, 0))] * 3,
        out_shape=[o64, o64, o64],
    )(x, Wq, bq.reshape(1, 64), Wk, bk.reshape(1, 64), Wv)


def _tc_layer1_tables(oa, ob, dparts, b0, Wq1b, bq1b, Wk1b, bk1b, W1p):
    blk = 2000

    def body(a_ref, b_ref, dp_ref, b0_ref, wq_ref, bq_ref, wk_ref, bk_ref,
             wv_ref, q_ref, k_ref, v_ref):
        d = dp_ref[0] + dp_ref[1] + 1e-16
        d64 = jnp.broadcast_to(d[:, :, None], (blk, 8, 8)).reshape(blk, 64)
        h = jax.nn.relu((a_ref[...] + b_ref[...]) / d64 + b0_ref[...])
        q_ref[...] = jax.nn.relu(
            jnp.dot(h, wq_ref[...], preferred_element_type=jnp.float32,
                    precision=_HIGH) + bq_ref[...])
        k_ref[...] = jax.nn.relu(
            jnp.dot(h, wk_ref[...], preferred_element_type=jnp.float32,
                    precision=_HIGH) + bk_ref[...])
        v_ref[...] = jnp.dot(h, wv_ref[...], preferred_element_type=jnp.float32,
                             precision=_HIGH)

    full = lambda i: (0, 0)
    o8 = jax.ShapeDtypeStruct((N, 8), jnp.float32)
    return pl.pallas_call(
        body,
        grid=(N // blk,),
        in_specs=[
            pl.BlockSpec((blk, 64), lambda i: (i, 0)),
            pl.BlockSpec((blk, 64), lambda i: (i, 0)),
            pl.BlockSpec((2, blk, 8), lambda i: (0, i, 0)),
            pl.BlockSpec((1, 64), full),
            pl.BlockSpec((64, 8), full),
            pl.BlockSpec((1, 8), full),
            pl.BlockSpec((64, 8), full),
            pl.BlockSpec((1, 8), full),
            pl.BlockSpec((64, 8), full),
        ],
        out_specs=[pl.BlockSpec((blk, 8), lambda i: (i, 0))] * 3,
        out_shape=[o8, o8, o8],
    )(oa, ob, dparts, b0.reshape(1, 64), Wq1b, bq1b, Wk1b, bk1b, W1p)


def _tc_final(o1a, o1b, dparts, b1p):
    blk = 2000

    def body(a_ref, b_ref, dp_ref, bias_ref, o_ref):
        d = dp_ref[0] + dp_ref[1] + 1e-16
        o_ref[...] = (a_ref[...] + b_ref[...]) / d + bias_ref[...]

    return pl.pallas_call(
        body,
        grid=(N // blk,),
        in_specs=[
            pl.BlockSpec((blk, 8), lambda i: (i, 0)),
            pl.BlockSpec((blk, 8), lambda i: (i, 0)),
            pl.BlockSpec((2, blk, 8), lambda i: (0, i, 0)),
            pl.BlockSpec((1, 8), lambda i: (0, 0)),
        ],
        out_specs=pl.BlockSpec((blk, 8), lambda i: (i, 0)),
        out_shape=jax.ShapeDtypeStruct((N, 8), jnp.float32),
    )(o1a, o1b, dparts, b1p)


def kernel(x, edge_index, Wq0, bq0, Wk0, bk0, W0, b0, Wq1, bq1, Wk1, bk1, W1, b1):
    loops = jnp.arange(N, dtype=jnp.int32)
    pad = jnp.zeros((E_PAD - E_TOT,), jnp.int32)
    row = jnp.concatenate([edge_index[0].astype(jnp.int32), loops, pad])
    col = jnp.concatenate([edge_index[1].astype(jnp.int32), loops, pad])
    row_s = jnp.concatenate([edge_index[0].astype(jnp.int32), loops,
                             jnp.full((E_PAD - E_TOT,), N, jnp.int32)])

    z8 = jnp.zeros((RPT, 8), jnp.float32)
    z64 = jnp.zeros((RPT, 64), jnp.float32)

    perm = (jnp.arange(64) % 8) * 8 + jnp.arange(64) // 8
    Q, K, V = _tc_qkv(x, Wq0[:, perm], bq0[perm], Wk0[:, perm], bk0[perm], W0)
    p, dpart = _sc_attn0(Q, K, row, row_s, col, z8)
    opart = _sc_aggr0(p, V, row_s, col, z64)

    Wq1b = jnp.broadcast_to(Wq1, (64, 8))
    bq1b = jnp.broadcast_to(bq1, (1, 8))
    Wk1b = jnp.broadcast_to(Wk1, (64, 8))
    bk1b = jnp.broadcast_to(bk1, (1, 8))
    W1p = jnp.pad(W1, ((0, 0), (0, 1)))
    q1t, k1t, v1t = _tc_layer1_tables(opart[0], opart[1], dpart, b0,
                                      Wq1b, bq1b, Wk1b, bk1b, W1p)
    p1, d1part = _sc_attn1(q1t, k1t, row, row_s, col, z8)
    o1part = _sc_aggr1(p1, v1t, row_s, col, z8)
    out8 = _tc_final(o1part[0], o1part[1], d1part,
                     jnp.pad(b1, (0, 1)).reshape(1, 8))
    return out8[:, :7]


# speedup vs baseline: 105.9022x; 1.1422x over previous
import jax
import jax.numpy as jnp
from jax import lax
from jax.experimental import pallas as pl
from jax.experimental.pallas import tpu as pltpu
from jax.experimental.pallas import tpu_sc as plsc

N = 10000
DF = 128
E_RAW = 320000
E_TOT = E_RAW + N
E_PAD = 331776
NW = 32
N_ACC = 10016
RPT = N_ACC // 16

_HIGH = lax.Precision.HIGHEST


def _mesh():
    return plsc.VectorSubcoreMesh(core_axis_name="c", subcore_axis_name="s")


_SC_PARAMS = pltpu.CompilerParams(use_tc_tiling_on_sc=False,
                                  needs_layout_passes=False)


def _vperm(v, idx):
    return lax.gather(
        v, idx.reshape(16, 1),
        lax.GatherDimensionNumbers(offset_dims=(), collapsed_slice_dims=(0,),
                                   start_index_map=(0,)),
        (1,), mode=lax.GatherScatterMode.PROMISE_IN_BOUNDS)


def _sc_layer0(q, k, v, row, row_s, col, z8, z64):
    ca = 144
    per_w = E_PAD // NW
    n_ch = per_w // ca

    i32 = jnp.int32
    f32 = jnp.float32
    scr = ([pltpu.VMEM((ca,), i32)] * 6 +
           [pltpu.VMEM((ca, 64), f32)] * 7 +
           [pltpu.VMEM((ca, 8), f32)] +
           [pltpu.VMEM_SHARED((N_ACC, 8), f32),
            pltpu.VMEM_SHARED((N_ACC, 64), f32)] +
           [pltpu.SemaphoreType.DMA] * 6)

    @pl.kernel(out_type=(jax.ShapeDtypeStruct((2, N_ACC, 64), f32),
                         jax.ShapeDtypeStruct((2, N_ACC, 8), f32)),
               mesh=_mesh(), compiler_params=_SC_PARAMS, scratch_types=scr)
    def l0_kernel(q_hbm, k_hbm, v_hbm, row_hbm, rs_hbm, col_hbm,
                  z8_hbm, z64_hbm, out_hbm, dh_hbm,
                  ir0, ir1, is0, is1, ic0, ic1,
                  qe0, qe1, ke0, ke1, ve0, ve1, ct_v, pv_v,
                  den, acc, sq0, sq1, sk0, sk1, sv0, sv1):
        c = lax.axis_index("c")
        s = lax.axis_index("s")
        base_w = (s * 2 + c) * per_w
        IR = (ir0, ir1)
        IS = (is0, is1)
        IC = (ic0, ic1)
        QE = (qe0, qe1)
        KE = (ke0, ke1)
        VE = (ve0, ve1)
        SQ = (sq0, sq1)
        SK = (sk0, sk1)
        SV = (sv0, sv1)
        io16 = lax.iota(i32, 16)
        p8x = io16 ^ 8
        h8 = io16 & 7
        lo8 = io16 < 8
        hi1 = jnp.where(io16 >= 8, 1, 0).astype(i32)

        rows = pl.ds(s * RPT, RPT)
        pltpu.sync_copy(z8_hbm, den.at[rows])
        pltpu.sync_copy(z64_hbm, acc.at[rows])
        plsc.subcore_barrier()

        def issue(b, ch):
            base = base_w + ch * ca
            pltpu.sync_copy(row_hbm.at[pl.ds(base, ca)], IR[b])
            pltpu.sync_copy(rs_hbm.at[pl.ds(base, ca)], IS[b])
            pltpu.sync_copy(col_hbm.at[pl.ds(base, ca)], IC[b])
            pltpu.async_copy(q_hbm.at[IR[b]], QE[b], SQ[b])
            pltpu.async_copy(k_hbm.at[IC[b]], KE[b], SK[b])
            pltpu.async_copy(v_hbm.at[IC[b]], VE[b], SV[b])

        for b in range(2):
            issue(b, b)

        @pl.loop(0, n_ch // 2)
        def _(gg):
            for b in range(2):
                ch = gg * 2 + b
                pltpu.make_async_copy(q_hbm.at[IR[b]], QE[b], SQ[b]).wait()
                pltpu.make_async_copy(k_hbm.at[IC[b]], KE[b], SK[b]).wait()
                pltpu.make_async_copy(v_hbm.at[IC[b]], VE[b], SV[b]).wait()

                qe_v, ke_v, ve_v = QE[b], KE[b], VE[b]

                @plsc.parallel_loop(0, ca, unroll=4)
                def _(e):
                    acc_r = None
                    for j in range(4):
                        qv = qe_v[e, pl.ds(j * 16, 16)]
                        kv = ke_v[e, pl.ds(j * 16, 16)]
                        pr = qv * kv
                        acc_r = pr if acc_r is None else acc_r + pr
                    att = acc_r + _vperm(acc_r, p8x)
                    t = jnp.exp(att)
                    plsc.store_scatter(pv_v, [jnp.full((16,), e, i32), h8],
                                       t, mask=lo8)
                    for j in range(4):
                        tj = _vperm(t, jnp.full((16,), 2 * j, i32) + hi1)
                        sl = pl.ds(j * 16, 16)
                        ct_v[e, sl] = ve_v[e, sl] * tj

                pltpu.sync_copy(pv_v, den.at[IS[b]], add=True)
                pltpu.sync_copy(ct_v, acc.at[IS[b]], add=True)

                @pl.when(ch + 2 < n_ch)
                def _():
                    issue(b, ch + 2)

        plsc.subcore_barrier()
        pltpu.sync_copy(den.at[rows], dh_hbm.at[c, rows])
        pltpu.sync_copy(acc.at[rows], out_hbm.at[c, rows])

    return l0_kernel(q, k, v, row, row_s, col, z8, z64)


def _sc_layer1(q1t, k1t, v1t, row, row_s, col, z8):
    ca = 648
    per_w = E_PAD // NW
    n_ch = per_w // ca

    i32 = jnp.int32
    f32 = jnp.float32
    scr = ([pltpu.VMEM((ca,), i32)] * 6 +
           [pltpu.VMEM((ca, 8), f32)] * 8 +
           [pltpu.VMEM_SHARED((N_ACC, 8), f32),
            pltpu.VMEM_SHARED((N_ACC, 8), f32)] +
           [pltpu.SemaphoreType.DMA] * 6)

    @pl.kernel(out_type=(jax.ShapeDtypeStruct((2, N_ACC, 8), f32),
                         jax.ShapeDtypeStruct((2, N_ACC, 8), f32)),
               mesh=_mesh(), compiler_params=_SC_PARAMS, scratch_types=scr)
    def l1_kernel(q_hbm, k_hbm, v_hbm, row_hbm, rs_hbm, col_hbm,
                  z8_hbm, out_hbm, dh_hbm,
                  ir0, ir1, is0, is1, ic0, ic1,
                  qe0, qe1, ke0, ke1, ve0, ve1, ct_v, pv_v,
                  den, acc, sq0, sq1, sk0, sk1, sv0, sv1):
        c = lax.axis_index("c")
        s = lax.axis_index("s")
        base_w = (s * 2 + c) * per_w
        IR = (ir0, ir1)
        IS = (is0, is1)
        IC = (ic0, ic1)
        QE = (qe0, qe1)
        KE = (ke0, ke1)
        VE = (ve0, ve1)
        SQ = (sq0, sq1)
        SK = (sk0, sk1)
        SV = (sv0, sv1)
        io16 = lax.iota(i32, 16)
        h8 = io16 & 7
        hi1 = jnp.where(io16 >= 8, 1, 0).astype(i32)

        rows = pl.ds(s * RPT, RPT)
        pltpu.sync_copy(z8_hbm, den.at[rows])
        pltpu.sync_copy(z8_hbm, acc.at[rows])
        plsc.subcore_barrier()

        def issue(b, ch):
            base = base_w + ch * ca
            pltpu.sync_copy(row_hbm.at[pl.ds(base, ca)], IR[b])
            pltpu.sync_copy(rs_hbm.at[pl.ds(base, ca)], IS[b])
            pltpu.sync_copy(col_hbm.at[pl.ds(base, ca)], IC[b])
            pltpu.async_copy(q_hbm.at[IR[b]], QE[b], SQ[b])
            pltpu.async_copy(k_hbm.at[IC[b]], KE[b], SK[b])
            pltpu.async_copy(v_hbm.at[IC[b]], VE[b], SV[b])

        for b in range(2):
            issue(b, b)

        @pl.loop(0, n_ch // 2)
        def _(gg):
            for b in range(2):
                ch = gg * 2 + b
                pltpu.make_async_copy(q_hbm.at[IR[b]], QE[b], SQ[b]).wait()
                pltpu.make_async_copy(k_hbm.at[IC[b]], KE[b], SK[b]).wait()
                pltpu.make_async_copy(v_hbm.at[IC[b]], VE[b], SV[b]).wait()

                qe_v, ke_v, ve_v = QE[b], KE[b], VE[b]

                @plsc.parallel_loop(0, ca // 2, unroll=8)
                def _(ee):
                    e2 = jnp.full((16,), 2 * ee, i32) + hi1
                    qv = plsc.load_gather(qe_v, [e2, h8])
                    kv = plsc.load_gather(ke_v, [e2, h8])
                    vv = plsc.load_gather(ve_v, [e2, h8])
                    t = jnp.exp(qv * kv)
                    plsc.store_scatter(pv_v, [e2, h8], t)
                    plsc.store_scatter(ct_v, [e2, h8], t * vv)

                pltpu.sync_copy(pv_v, den.at[IS[b]], add=True)
                pltpu.sync_copy(ct_v, acc.at[IS[b]], add=True)

                @pl.when(ch + 2 < n_ch)
                def _():
                    issue(b, ch + 2)

        plsc.subcore_barrier()
        pltpu.sync_copy(den.at[rows], dh_hbm.at[c, rows])
        pltpu.sync_copy(acc.at[rows], out_hbm.at[c, rows])

    return l1_kernel(q1t, k1t, v1t, row, row_s, col, z8)


def _tc_qkv(x, Wq, bq, Wk, bk, Wv):
    blk = 2000

    def body(x_ref, wq_ref, bq_ref, wk_ref, bk_ref, wv_ref, q_ref, k_ref, v_ref):
        xb = x_ref[...]
        q_ref[...] = jax.nn.relu(
            jnp.dot(xb, wq_ref[...], preferred_element_type=jnp.float32,
                    precision=_HIGH) + bq_ref[...])
        k_ref[...] = jax.nn.relu(
            jnp.dot(xb, wk_ref[...], preferred_element_type=jnp.float32,
                    precision=_HIGH) + bk_ref[...])
        v_ref[...] = jnp.dot(xb, wv_ref[...], preferred_element_type=jnp.float32,
                             precision=_HIGH)

    full = lambda i: (0, 0)
    o64 = jax.ShapeDtypeStruct((N, 64), jnp.float32)
    return pl.pallas_call(
        body,
        grid=(N // blk,),
        in_specs=[
            pl.BlockSpec((blk, DF), lambda i: (i, 0)),
            pl.BlockSpec((DF, 64), full),
            pl.BlockSpec((1, 64), full),
            pl.BlockSpec((DF, 64), full),
            pl.BlockSpec((1, 64), full),
            pl.BlockSpec((DF, 64), full),
        ],
        out_specs=[pl.BlockSpec((blk, 64), lambda i: (i, 0))] * 3,
        out_shape=[o64, o64, o64],
    )(x, Wq, bq.reshape(1, 64), Wk, bk.reshape(1, 64), Wv)


def _tc_layer1_tables(oa, ob, dparts, b0, Wq1b, bq1b, Wk1b, bk1b, W1p):
    blk = 2000

    def body(a_ref, b_ref, dp_ref, b0_ref, wq_ref, bq_ref, wk_ref, bk_ref,
             wv_ref, q_ref, k_ref, v_ref):
        d = dp_ref[0] + dp_ref[1] + 1e-16
        d64 = jnp.broadcast_to(d[:, :, None], (blk, 8, 8)).reshape(blk, 64)
        h = jax.nn.relu((a_ref[...] + b_ref[...]) / d64 + b0_ref[...])
        q_ref[...] = jax.nn.relu(
            jnp.dot(h, wq_ref[...], preferred_element_type=jnp.float32,
                    precision=_HIGH) + bq_ref[...])
        k_ref[...] = jax.nn.relu(
            jnp.dot(h, wk_ref[...], preferred_element_type=jnp.float32,
                    precision=_HIGH) + bk_ref[...])
        v_ref[...] = jnp.dot(h, wv_ref[...], preferred_element_type=jnp.float32,
                             precision=_HIGH)

    full = lambda i: (0, 0)
    o8 = jax.ShapeDtypeStruct((N, 8), jnp.float32)
    return pl.pallas_call(
        body,
        grid=(N // blk,),
        in_specs=[
            pl.BlockSpec((blk, 64), lambda i: (i, 0)),
            pl.BlockSpec((blk, 64), lambda i: (i, 0)),
            pl.BlockSpec((2, blk, 8), lambda i: (0, i, 0)),
            pl.BlockSpec((1, 64), full),
            pl.BlockSpec((64, 8), full),
            pl.BlockSpec((1, 8), full),
            pl.BlockSpec((64, 8), full),
            pl.BlockSpec((1, 8), full),
            pl.BlockSpec((64, 8), full),
        ],
        out_specs=[pl.BlockSpec((blk, 8), lambda i: (i, 0))] * 3,
        out_shape=[o8, o8, o8],
    )(oa, ob, dparts, b0.reshape(1, 64), Wq1b, bq1b, Wk1b, bk1b, W1p)


def _tc_final(o1a, o1b, dparts, b1p):
    blk = 2000

    def body(a_ref, b_ref, dp_ref, bias_ref, o_ref):
        d = dp_ref[0] + dp_ref[1] + 1e-16
        o_ref[...] = (a_ref[...] + b_ref[...]) / d + bias_ref[...]

    return pl.pallas_call(
        body,
        grid=(N // blk,),
        in_specs=[
            pl.BlockSpec((blk, 8), lambda i: (i, 0)),
            pl.BlockSpec((blk, 8), lambda i: (i, 0)),
            pl.BlockSpec((2, blk, 8), lambda i: (0, i, 0)),
            pl.BlockSpec((1, 8), lambda i: (0, 0)),
        ],
        out_specs=pl.BlockSpec((blk, 8), lambda i: (i, 0)),
        out_shape=jax.ShapeDtypeStruct((N, 8), jnp.float32),
    )(o1a, o1b, dparts, b1p)


def kernel(x, edge_index, Wq0, bq0, Wk0, bk0, W0, b0, Wq1, bq1, Wk1, bk1, W1, b1):
    loops = jnp.arange(N, dtype=jnp.int32)
    pad = jnp.zeros((E_PAD - E_TOT,), jnp.int32)
    row = jnp.concatenate([edge_index[0].astype(jnp.int32), loops, pad])
    col = jnp.concatenate([edge_index[1].astype(jnp.int32), loops, pad])
    row_s = jnp.concatenate([edge_index[0].astype(jnp.int32), loops,
                             jnp.full((E_PAD - E_TOT,), N, jnp.int32)])

    z8 = jnp.zeros((RPT, 8), jnp.float32)
    z64 = jnp.zeros((RPT, 64), jnp.float32)

    perm = (jnp.arange(64) % 8) * 8 + jnp.arange(64) // 8
    Q, K, V = _tc_qkv(x, Wq0[:, perm], bq0[perm], Wk0[:, perm], bk0[perm], W0)
    opart, dpart = _sc_layer0(Q, K, V, row, row_s, col, z8, z64)

    Wq1b = jnp.broadcast_to(Wq1, (64, 8))
    bq1b = jnp.broadcast_to(bq1, (1, 8))
    Wk1b = jnp.broadcast_to(Wk1, (64, 8))
    bk1b = jnp.broadcast_to(bk1, (1, 8))
    W1p = jnp.pad(W1, ((0, 0), (0, 1)))
    q1t, k1t, v1t = _tc_layer1_tables(opart[0], opart[1], dpart, b0,
                                      Wq1b, bq1b, Wk1b, bk1b, W1p)
    o1part, d1part = _sc_layer1(q1t, k1t, v1t, row, row_s, col, z8)
    out8 = _tc_final(o1part[0], o1part[1], d1part,
                     jnp.pad(b1, (0, 1)).reshape(1, 8))
    return out8[:, :7]


# speedup vs baseline: 106.4999x; 1.0056x over previous
import jax
import jax.numpy as jnp
from jax import lax
from jax.experimental import pallas as pl
from jax.experimental.pallas import tpu as pltpu
from jax.experimental.pallas import tpu_sc as plsc

N = 10000
DF = 128
E_RAW = 320000
E_TOT = E_RAW + N
E_PAD = 331776
NW = 32
N_ACC = 10016
RPT = N_ACC // 16

_HIGH = lax.Precision.HIGHEST


def _mesh():
    return plsc.VectorSubcoreMesh(core_axis_name="c", subcore_axis_name="s")


_SC_PARAMS = pltpu.CompilerParams(use_tc_tiling_on_sc=False,
                                  needs_layout_passes=False)


def _vperm(v, idx):
    return lax.gather(
        v, idx.reshape(16, 1),
        lax.GatherDimensionNumbers(offset_dims=(), collapsed_slice_dims=(0,),
                                   start_index_map=(0,)),
        (1,), mode=lax.GatherScatterMode.PROMISE_IN_BOUNDS)


def _sc_layer0(q, k, v, row, row_s, col, z8, z64):
    ca = 144
    per_w = E_PAD // NW
    n_ch = per_w // ca

    i32 = jnp.int32
    f32 = jnp.float32
    scr = ([pltpu.VMEM((ca,), i32)] * 6 +
           [pltpu.VMEM((ca, 64), f32)] * 7 +
           [pltpu.VMEM((ca, 8), f32)] +
           [pltpu.VMEM_SHARED((N_ACC, 8), f32),
            pltpu.VMEM_SHARED((N_ACC, 64), f32)] +
           [pltpu.SemaphoreType.DMA] * 6)

    @pl.kernel(out_type=(jax.ShapeDtypeStruct((2, N_ACC, 64), f32),
                         jax.ShapeDtypeStruct((2, N_ACC, 8), f32)),
               mesh=_mesh(), compiler_params=_SC_PARAMS, scratch_types=scr)
    def l0_kernel(q_hbm, k_hbm, v_hbm, row_hbm, rs_hbm, col_hbm,
                  z8_hbm, z64_hbm, out_hbm, dh_hbm,
                  ir0, ir1, is0, is1, ic0, ic1,
                  qe0, qe1, ke0, ke1, ve0, ve1, ct_v, pv_v,
                  den, acc, sq0, sq1, sk0, sk1, sv0, sv1):
        c = lax.axis_index("c")
        s = lax.axis_index("s")
        base_w = (s * 2 + c) * per_w
        IR = (ir0, ir1)
        IS = (is0, is1)
        IC = (ic0, ic1)
        QE = (qe0, qe1)
        KE = (ke0, ke1)
        VE = (ve0, ve1)
        SQ = (sq0, sq1)
        SK = (sk0, sk1)
        SV = (sv0, sv1)
        io16 = lax.iota(i32, 16)
        p8x = io16 ^ 8
        h8 = io16 & 7
        lo8 = io16 < 8
        hi1 = jnp.where(io16 >= 8, 1, 0).astype(i32)

        rows = pl.ds(s * RPT, RPT)
        pltpu.sync_copy(z8_hbm, den.at[rows])
        pltpu.sync_copy(z64_hbm, acc.at[rows])
        plsc.subcore_barrier()

        def issue(b, ch):
            base = base_w + ch * ca
            pltpu.sync_copy(row_hbm.at[pl.ds(base, ca)], IR[b])
            pltpu.sync_copy(rs_hbm.at[pl.ds(base, ca)], IS[b])
            pltpu.sync_copy(col_hbm.at[pl.ds(base, ca)], IC[b])
            pltpu.async_copy(q_hbm.at[IR[b]], QE[b], SQ[b])
            pltpu.async_copy(k_hbm.at[IC[b]], KE[b], SK[b])
            pltpu.async_copy(v_hbm.at[IC[b]], VE[b], SV[b])

        for b in range(2):
            issue(b, b)

        @pl.loop(0, n_ch // 2)
        def _(gg):
            for b in range(2):
                ch = gg * 2 + b
                pltpu.make_async_copy(q_hbm.at[IR[b]], QE[b], SQ[b]).wait()
                pltpu.make_async_copy(k_hbm.at[IC[b]], KE[b], SK[b]).wait()
                pltpu.make_async_copy(v_hbm.at[IC[b]], VE[b], SV[b]).wait()

                qe_v, ke_v, ve_v = QE[b], KE[b], VE[b]

                @plsc.parallel_loop(0, ca, unroll=4)
                def _(e):
                    acc_r = None
                    for j in range(4):
                        qv = qe_v[e, pl.ds(j * 16, 16)]
                        kv = ke_v[e, pl.ds(j * 16, 16)]
                        pr = qv * kv
                        acc_r = pr if acc_r is None else acc_r + pr
                    att = acc_r + _vperm(acc_r, p8x)
                    t = jnp.exp(att)
                    plsc.store_scatter(pv_v, [jnp.full((16,), e, i32), h8],
                                       t, mask=lo8)
                    for j in range(4):
                        tj = _vperm(t, jnp.full((16,), 2 * j, i32) + hi1)
                        sl = pl.ds(j * 16, 16)
                        ct_v[e, sl] = ve_v[e, sl] * tj

                pltpu.sync_copy(pv_v, den.at[IS[b]], add=True)
                pltpu.sync_copy(ct_v, acc.at[IS[b]], add=True)

                @pl.when(ch + 2 < n_ch)
                def _():
                    issue(b, ch + 2)

        plsc.subcore_barrier()
        pltpu.sync_copy(den.at[rows], dh_hbm.at[c, rows])
        pltpu.sync_copy(acc.at[rows], out_hbm.at[c, rows])

    return l0_kernel(q, k, v, row, row_s, col, z8, z64)


def _sc_layer1(q1t, k1t, v1t, row, row_s, col, z8):
    ca = 1296
    per_w = E_PAD // NW
    n_ch = per_w // ca

    i32 = jnp.int32
    f32 = jnp.float32
    scr = ([pltpu.VMEM((ca,), i32)] * 6 +
           [pltpu.VMEM((ca, 8), f32)] * 8 +
           [pltpu.VMEM_SHARED((N_ACC, 8), f32),
            pltpu.VMEM_SHARED((N_ACC, 8), f32)] +
           [pltpu.SemaphoreType.DMA] * 6)

    @pl.kernel(out_type=(jax.ShapeDtypeStruct((2, N_ACC, 8), f32),
                         jax.ShapeDtypeStruct((2, N_ACC, 8), f32)),
               mesh=_mesh(), compiler_params=_SC_PARAMS, scratch_types=scr)
    def l1_kernel(q_hbm, k_hbm, v_hbm, row_hbm, rs_hbm, col_hbm,
                  z8_hbm, out_hbm, dh_hbm,
                  ir0, ir1, is0, is1, ic0, ic1,
                  qe0, qe1, ke0, ke1, ve0, ve1, ct_v, pv_v,
                  den, acc, sq0, sq1, sk0, sk1, sv0, sv1):
        c = lax.axis_index("c")
        s = lax.axis_index("s")
        base_w = (s * 2 + c) * per_w
        IR = (ir0, ir1)
        IS = (is0, is1)
        IC = (ic0, ic1)
        QE = (qe0, qe1)
        KE = (ke0, ke1)
        VE = (ve0, ve1)
        SQ = (sq0, sq1)
        SK = (sk0, sk1)
        SV = (sv0, sv1)
        io16 = lax.iota(i32, 16)
        h8 = io16 & 7
        hi1 = jnp.where(io16 >= 8, 1, 0).astype(i32)

        rows = pl.ds(s * RPT, RPT)
        pltpu.sync_copy(z8_hbm, den.at[rows])
        pltpu.sync_copy(z8_hbm, acc.at[rows])
        plsc.subcore_barrier()

        def issue(b, ch):
            base = base_w + ch * ca
            pltpu.sync_copy(row_hbm.at[pl.ds(base, ca)], IR[b])
            pltpu.sync_copy(rs_hbm.at[pl.ds(base, ca)], IS[b])
            pltpu.sync_copy(col_hbm.at[pl.ds(base, ca)], IC[b])
            pltpu.async_copy(q_hbm.at[IR[b]], QE[b], SQ[b])
            pltpu.async_copy(k_hbm.at[IC[b]], KE[b], SK[b])
            pltpu.async_copy(v_hbm.at[IC[b]], VE[b], SV[b])

        for b in range(2):
            issue(b, b)

        @pl.loop(0, n_ch // 2)
        def _(gg):
            for b in range(2):
                ch = gg * 2 + b
                pltpu.make_async_copy(q_hbm.at[IR[b]], QE[b], SQ[b]).wait()
                pltpu.make_async_copy(k_hbm.at[IC[b]], KE[b], SK[b]).wait()
                pltpu.make_async_copy(v_hbm.at[IC[b]], VE[b], SV[b]).wait()

                qe_v, ke_v, ve_v = QE[b], KE[b], VE[b]

                @plsc.parallel_loop(0, ca // 2, unroll=8)
                def _(ee):
                    e2 = jnp.full((16,), 2 * ee, i32) + hi1
                    qv = plsc.load_gather(qe_v, [e2, h8])
                    kv = plsc.load_gather(ke_v, [e2, h8])
                    vv = plsc.load_gather(ve_v, [e2, h8])
                    t = jnp.exp(qv * kv)
                    plsc.store_scatter(pv_v, [e2, h8], t)
                    plsc.store_scatter(ct_v, [e2, h8], t * vv)

                pltpu.sync_copy(pv_v, den.at[IS[b]], add=True)
                pltpu.sync_copy(ct_v, acc.at[IS[b]], add=True)

                @pl.when(ch + 2 < n_ch)
                def _():
                    issue(b, ch + 2)

        plsc.subcore_barrier()
        pltpu.sync_copy(den.at[rows], dh_hbm.at[c, rows])
        pltpu.sync_copy(acc.at[rows], out_hbm.at[c, rows])

    return l1_kernel(q1t, k1t, v1t, row, row_s, col, z8)


def _tc_qkv(x, Wq, bq, Wk, bk, Wv):
    blk = 2000

    def body(x_ref, wq_ref, bq_ref, wk_ref, bk_ref, wv_ref, q_ref, k_ref, v_ref):
        xb = x_ref[...]
        q_ref[...] = jax.nn.relu(
            jnp.dot(xb, wq_ref[...], preferred_element_type=jnp.float32,
                    precision=_HIGH) + bq_ref[...])
        k_ref[...] = jax.nn.relu(
            jnp.dot(xb, wk_ref[...], preferred_element_type=jnp.float32,
                    precision=_HIGH) + bk_ref[...])
        v_ref[...] = jnp.dot(xb, wv_ref[...], preferred_element_type=jnp.float32,
                             precision=_HIGH)

    full = lambda i: (0, 0)
    o64 = jax.ShapeDtypeStruct((N, 64), jnp.float32)
    return pl.pallas_call(
        body,
        grid=(N // blk,),
        in_specs=[
            pl.BlockSpec((blk, DF), lambda i: (i, 0)),
            pl.BlockSpec((DF, 64), full),
            pl.BlockSpec((1, 64), full),
            pl.BlockSpec((DF, 64), full),
            pl.BlockSpec((1, 64), full),
            pl.BlockSpec((DF, 64), full),
        ],
        out_specs=[pl.BlockSpec((blk, 64), lambda i: (i, 0))] * 3,
        out_shape=[o64, o64, o64],
    )(x, Wq, bq.reshape(1, 64), Wk, bk.reshape(1, 64), Wv)


def _tc_layer1_tables(oa, ob, dparts, b0, Wq1b, bq1b, Wk1b, bk1b, W1p):
    blk = 2000

    def body(a_ref, b_ref, dp_ref, b0_ref, wq_ref, bq_ref, wk_ref, bk_ref,
             wv_ref, q_ref, k_ref, v_ref):
        d = dp_ref[0] + dp_ref[1] + 1e-16
        d64 = jnp.broadcast_to(d[:, :, None], (blk, 8, 8)).reshape(blk, 64)
        h = jax.nn.relu((a_ref[...] + b_ref[...]) / d64 + b0_ref[...])
        q_ref[...] = jax.nn.relu(
            jnp.dot(h, wq_ref[...], preferred_element_type=jnp.float32,
                    precision=_HIGH) + bq_ref[...])
        k_ref[...] = jax.nn.relu(
            jnp.dot(h, wk_ref[...], preferred_element_type=jnp.float32,
                    precision=_HIGH) + bk_ref[...])
        v_ref[...] = jnp.dot(h, wv_ref[...], preferred_element_type=jnp.float32,
                             precision=_HIGH)

    full = lambda i: (0, 0)
    o8 = jax.ShapeDtypeStruct((N, 8), jnp.float32)
    return pl.pallas_call(
        body,
        grid=(N // blk,),
        in_specs=[
            pl.BlockSpec((blk, 64), lambda i: (i, 0)),
            pl.BlockSpec((blk, 64), lambda i: (i, 0)),
            pl.BlockSpec((2, blk, 8), lambda i: (0, i, 0)),
            pl.BlockSpec((1, 64), full),
            pl.BlockSpec((64, 8), full),
            pl.BlockSpec((1, 8), full),
            pl.BlockSpec((64, 8), full),
            pl.BlockSpec((1, 8), full),
            pl.BlockSpec((64, 8), full),
        ],
        out_specs=[pl.BlockSpec((blk, 8), lambda i: (i, 0))] * 3,
        out_shape=[o8, o8, o8],
    )(oa, ob, dparts, b0.reshape(1, 64), Wq1b, bq1b, Wk1b, bk1b, W1p)


def _tc_final(o1a, o1b, dparts, b1p):
    blk = 2000

    def body(a_ref, b_ref, dp_ref, bias_ref, o_ref):
        d = dp_ref[0] + dp_ref[1] + 1e-16
        o_ref[...] = (a_ref[...] + b_ref[...]) / d + bias_ref[...]

    return pl.pallas_call(
        body,
        grid=(N // blk,),
        in_specs=[
            pl.BlockSpec((blk, 8), lambda i: (i, 0)),
            pl.BlockSpec((blk, 8), lambda i: (i, 0)),
            pl.BlockSpec((2, blk, 8), lambda i: (0, i, 0)),
            pl.BlockSpec((1, 8), lambda i: (0, 0)),
        ],
        out_specs=pl.BlockSpec((blk, 8), lambda i: (i, 0)),
        out_shape=jax.ShapeDtypeStruct((N, 8), jnp.float32),
    )(o1a, o1b, dparts, b1p)


def kernel(x, edge_index, Wq0, bq0, Wk0, bk0, W0, b0, Wq1, bq1, Wk1, bk1, W1, b1):
    loops = jnp.arange(N, dtype=jnp.int32)
    pad = jnp.zeros((E_PAD - E_TOT,), jnp.int32)
    row = jnp.concatenate([edge_index[0].astype(jnp.int32), loops, pad])
    col = jnp.concatenate([edge_index[1].astype(jnp.int32), loops, pad])
    row_s = jnp.concatenate([edge_index[0].astype(jnp.int32), loops,
                             jnp.full((E_PAD - E_TOT,), N, jnp.int32)])

    z8 = jnp.zeros((RPT, 8), jnp.float32)
    z64 = jnp.zeros((RPT, 64), jnp.float32)

    perm = (jnp.arange(64) % 8) * 8 + jnp.arange(64) // 8
    Q, K, V = _tc_qkv(x, Wq0[:, perm], bq0[perm], Wk0[:, perm], bk0[perm], W0)
    opart, dpart = _sc_layer0(Q, K, V, row, row_s, col, z8, z64)

    Wq1b = jnp.broadcast_to(Wq1, (64, 8))
    bq1b = jnp.broadcast_to(bq1, (1, 8))
    Wk1b = jnp.broadcast_to(Wk1, (64, 8))
    bk1b = jnp.broadcast_to(bk1, (1, 8))
    W1p = jnp.pad(W1, ((0, 0), (0, 1)))
    q1t, k1t, v1t = _tc_layer1_tables(opart[0], opart[1], dpart, b0,
                                      Wq1b, bq1b, Wk1b, bk1b, W1p)
    o1part, d1part = _sc_layer1(q1t, k1t, v1t, row, row_s, col, z8)
    out8 = _tc_final(o1part[0], o1part[1], d1part,
                     jnp.pad(b1, (0, 1)).reshape(1, 8))
    return out8[:, :7]
